# Initial kernel scaffold; baseline (speedup 1.0000x reference)
import jax
import jax.numpy as jnp
from jax.experimental import pallas as pl


def kernel(x_s, x_t, edge_index, edge_attr, x_u, params):
    raise NotImplementedError("write your pallas kernel here")


# speedup vs baseline: 2.2010x; 2.2010x over previous
import functools

import jax
import jax.numpy as jnp
from jax import lax
from jax.experimental import pallas as pl
from jax.experimental.pallas import tpu as pltpu
from jax.experimental.pallas import tpu_sc as plsc

_N = 10000
_E = 160000
_D = 128
_SLOPE = 0.01
_NC, _NS, _L = 2, 16, 16
_NW = _NC * _NS

_CE = 80
_EPT = _E // _NS
_NCHS = _EPT // _CE
_IPAD = 128
_CZ = 80
_NZ = _N // _CZ

f32 = jnp.float32


def _lrelu(x):
    return jnp.where(x >= 0, x, _SLOPE * x)


def _mesh():
    return plsc.VectorSubcoreMesh(core_axis_name="c", subcore_axis_name="s")


def _fill(buf, nrows, ncols, val):
    def row(r, _):
        for g in range(ncols // _L):
            buf[r, pl.ds(g * _L, _L)] = jnp.full((_L,), val, f32)
        return 0
    lax.fori_loop(0, nrows, row, 0)


def _rr_chunks(sid, fn):
    for k in range(-(-_NZ // _NS)):
        j = sid + _NS * k
        pl.when(j < _NZ)(functools.partial(fn, j))


def _build_gather(n_tab):
    ew = _E // _NW
    c = 200
    nch = ew // c

    def body(*refs):
        tabs = refs[:n_tab]
        idxs = refs[n_tab:2 * n_tab]
        outs = refs[2 * n_tab:3 * n_tab]
        idx_v, rows_v, sem = refs[3 * n_tab:]
        wid = lax.axis_index("s") * _NC + lax.axis_index("c")
        base = wid * ew
        for tab, ih, oh in zip(tabs, idxs, outs):
            pltpu.sync_copy(ih.at[pl.ds(base, ew)], idx_v)

            def step(k, _):
                pltpu.async_copy(tab.at[idx_v.at[pl.ds(k * c, c)]], rows_v,
                                 sem).wait()
                pltpu.sync_copy(rows_v, oh.at[pl.ds(base + k * c, c)])
                return 0
            lax.fori_loop(0, nch, step, 0)

    return pl.kernel(
        body,
        out_type=[jax.ShapeDtypeStruct((_E, _D), f32)] * n_tab,
        mesh=_mesh(),
        scratch_types=[
            pltpu.VMEM((ew,), jnp.int32),
            pltpu.VMEM((c, _D), f32),
            pltpu.SemaphoreType.DMA,
        ],
    )


def _stats_body(msg_h, srcp_h, sum_h, sq_h, cnt_h, idx_v, mbuf, sbuf, acc):
    cid = lax.axis_index("c")
    sid = lax.axis_index("s")
    pltpu.sync_copy(srcp_h.at[pl.ds(sid * _IPAD, _IPAD)], idx_v)

    def one_pass(f0, kind):
        _fill(mbuf, _CZ, 128, 0.0)

        def zero(j):
            pltpu.sync_copy(mbuf, acc.at[pl.ds(j * _CZ, _CZ)])
        _rr_chunks(sid, zero)
        if kind == "cnt":
            _fill(sbuf, _CE, 128, 1.0)
        plsc.subcore_barrier()

        def step(j, _):
            idxr = idx_v.at[j]
            if kind == "cnt":
                pltpu.sync_copy(sbuf, acc.at[idxr], add=True)
                return 0
            e0 = sid * _EPT + j * _CE
            pltpu.sync_copy(msg_h.at[pl.ds(e0, _CE), pl.ds(f0, 128)], mbuf)
            if kind == "sq":
                def sqrow(r, _):
                    for g in range(128 // _L):
                        v = mbuf[r, pl.ds(g * _L, _L)]
                        sbuf[r, pl.ds(g * _L, _L)] = v * v
                    return 0
                lax.fori_loop(0, _CE, sqrow, 0)
                pltpu.sync_copy(sbuf, acc.at[idxr], add=True)
            else:
                pltpu.sync_copy(mbuf, acc.at[idxr], add=True)
            return 0
        lax.fori_loop(0, _NCHS, step, 0)
        plsc.subcore_barrier()

        out_h = {"sum": sum_h, "sq": sq_h, "cnt": cnt_h}[kind]

        def read(j):
            r0 = j * _CZ
            pltpu.sync_copy(acc.at[pl.ds(r0, _CZ)], mbuf)
            if kind == "cnt":
                pltpu.sync_copy(mbuf, out_h.at[pl.ds(r0, _CZ)])
            else:
                pltpu.sync_copy(mbuf,
                                out_h.at[pl.ds(r0, _CZ), pl.ds(f0, 128)])
        _rr_chunks(sid, read)
        plsc.subcore_barrier()

    for f0, core, kind in ((0, 0, "sum"), (128, 0, "sum"), (0, 0, "cnt"),
                           (0, 1, "sq"), (128, 1, "sq")):
        pl.when(cid == core)(functools.partial(one_pass, f0, kind))


def _build_stats():
    return pl.kernel(
        _stats_body,
        out_type=[jax.ShapeDtypeStruct((_N, 256), f32),
                  jax.ShapeDtypeStruct((_N, 256), f32),
                  jax.ShapeDtypeStruct((_N, 128), f32)],
        mesh=_mesh(),
        scratch_types=[
            pltpu.VMEM((_IPAD, _CE), jnp.int32),
            pltpu.VMEM((_CE, 128), f32),
            pltpu.VMEM((_CE, 128), f32),
            pltpu.VMEM_SHARED((_N, 128), f32),
        ],
    )


def _mom2_body(msg_h, srcp_h, ma_h, mb_h, s3_h, s4_h,
               idx_v, mbuf, gbuf, sem, acc):
    cid = lax.axis_index("c")
    sid = lax.axis_index("s")
    pltpu.sync_copy(srcp_h.at[pl.ds(sid * _IPAD, _IPAD)], idx_v)

    def one_pass(f0, mean_h, out_h, quart):
        _fill(mbuf, _CZ, 128, 0.0)

        def zero(j):
            pltpu.sync_copy(mbuf, acc.at[pl.ds(j * _CZ, _CZ)])
        _rr_chunks(sid, zero)
        plsc.subcore_barrier()

        def step(j, _):
            e0 = sid * _EPT + j * _CE
            idxr = idx_v.at[j]
            pltpu.sync_copy(msg_h.at[pl.ds(e0, _CE), pl.ds(f0, 128)], mbuf)
            pltpu.async_copy(mean_h.at[idxr], gbuf, sem).wait()

            def cube(r, _):
                for g in range(128 // _L):
                    sl = pl.ds(g * _L, _L)
                    d = mbuf[r, sl] - gbuf[r, sl]
                    if quart:
                        d2 = d * d
                        mbuf[r, sl] = d2 * d2
                    else:
                        mbuf[r, sl] = d * d * d
                return 0
            lax.fori_loop(0, _CE, cube, 0)
            pltpu.sync_copy(mbuf, acc.at[idxr], add=True)
            return 0
        lax.fori_loop(0, _NCHS, step, 0)
        plsc.subcore_barrier()

        def read(j):
            r0 = j * _CZ
            pltpu.sync_copy(acc.at[pl.ds(r0, _CZ)], mbuf)
            pltpu.sync_copy(mbuf, out_h.at[pl.ds(r0, _CZ), pl.ds(f0, 128)])
        _rr_chunks(sid, read)
        plsc.subcore_barrier()

    for f0, core, quart in ((0, 0, False), (128, 0, False),
                            (0, 1, True), (128, 1, True)):
        mh = ma_h if f0 == 0 else mb_h
        oh = s4_h if quart else s3_h
        pl.when(cid == core)(functools.partial(one_pass, f0, mh, oh, quart))


def _build_mom2():
    return pl.kernel(
        _mom2_body,
        out_type=[jax.ShapeDtypeStruct((_N, 256), f32),
                  jax.ShapeDtypeStruct((_N, 256), f32)],
        mesh=_mesh(),
        scratch_types=[
            pltpu.VMEM((_IPAD, _CE), jnp.int32),
            pltpu.VMEM((_CE, 128), f32),
            pltpu.VMEM((_CE, 128), f32),
            pltpu.SemaphoreType.DMA,
            pltpu.VMEM_SHARED((_N, 128), f32),
        ],
    )


def _agg_body(msg_h, tgtp_h, agg_h, idx_v, mbuf, acc):
    cid = lax.axis_index("c")
    sid = lax.axis_index("s")
    pltpu.sync_copy(tgtp_h.at[pl.ds(sid * _IPAD, _IPAD)], idx_v)

    def one_pass(f0):
        _fill(mbuf, _CZ, 128, 0.0)

        def zero(j):
            pltpu.sync_copy(mbuf, acc.at[pl.ds(j * _CZ, _CZ)])
        _rr_chunks(sid, zero)
        plsc.subcore_barrier()

        def step(j, _):
            e0 = sid * _EPT + j * _CE
            pltpu.sync_copy(msg_h.at[pl.ds(e0, _CE), pl.ds(f0, 128)], mbuf)
            pltpu.sync_copy(mbuf, acc.at[idx_v.at[j]], add=True)
            return 0
        lax.fori_loop(0, _NCHS, step, 0)
        plsc.subcore_barrier()

        def read(j):
            r0 = j * _CZ
            pltpu.sync_copy(acc.at[pl.ds(r0, _CZ)], mbuf)
            pltpu.sync_copy(mbuf, agg_h.at[pl.ds(r0, _CZ), pl.ds(f0, 128)])
        _rr_chunks(sid, read)
        plsc.subcore_barrier()

    pl.when(cid == 0)(functools.partial(one_pass, 0))
    pl.when(cid == 1)(functools.partial(one_pass, 128))


def _build_agg():
    return pl.kernel(
        _agg_body,
        out_type=[jax.ShapeDtypeStruct((_N, 256), f32)],
        mesh=_mesh(),
        scratch_types=[
            pltpu.VMEM((_IPAD, _CE), jnp.int32),
            pltpu.VMEM((_CE, 128), f32),
            pltpu.VMEM_SHARED((_N, 128), f32),
        ],
    )


_BE = 2000
_GE = _E // _BE
_BN = 1000
_GN = _N // _BN


def _full(shape):
    return pl.BlockSpec(shape, lambda i: (0,) * len(shape))


def _rows(b, w):
    return pl.BlockSpec((b, w), lambda i: (i, 0))


def _dot(a, b):
    return jnp.dot(a, b, preferred_element_type=f32)


def _acc_stats(y, ss_r, sq_r):
    @pl.when(pl.program_id(0) == 0)
    def _():
        ss_r[...] = jnp.zeros_like(ss_r)
        sq_r[...] = jnp.zeros_like(sq_r)
    ss_r[...] += jnp.sum(y, 0, keepdims=True)
    sq_r[...] += jnp.sum(y * y, 0, keepdims=True)


def _tc_edge_mlp(gs, gt, ea, xu, w1a, w1b, w1c, w1d, b1, w2, b2):
    def body(gs_r, gt_r, ea_r, xu_r, a_r, br_r, c_r, d_r, b1_r, w2_r, b2_r,
             h2_r, ss_r, sq_r):
        h = _dot(gs_r[...], a_r[...]) + _dot(gt_r[...], br_r[...])
        h += _dot(ea_r[...], c_r[...])
        h += _dot(xu_r[...], d_r[...]) + b1_r[...]
        y = _dot(_lrelu(h), w2_r[...]) + b2_r[...]
        h2_r[...] = y
        _acc_stats(y, ss_r, sq_r)

    return pl.pallas_call(
        body, grid=(_GE,),
        in_specs=[_rows(_BE, _D)] * 3 + [_full(xu.shape), _full(w1a.shape),
                  _full(w1b.shape), _full(w1c.shape), _full(w1d.shape),
                  _full(b1.shape), _full(w2.shape), _full(b2.shape)],
        out_specs=[_rows(_BE, _D), _full((1, _D)), _full((1, _D))],
        out_shape=[jax.ShapeDtypeStruct((_E, _D), f32),
                   jax.ShapeDtypeStruct((1, _D), f32),
                   jax.ShapeDtypeStruct((1, _D), f32)],
    )(gs, gt, ea, xu, w1a, w1b, w1c, w1d, b1, w2, b2)


def _tc_bn_edge_msg(h2, gt, ss, sq, g, b, w1a, w1b, b1, w2, b2):
    def body(h2_r, gt_r, ss_r, sq_r, g_r, b_r, a_r, br_r, b1_r, w2_r, b2_r,
             ean_r, msg_r):
        m = ss_r[...] * (1.0 / _E)
        v = sq_r[...] * (1.0 / _E) - m * m
        inv = lax.rsqrt(v + 1e-5)
        ean = (h2_r[...] - m) * inv * g_r[...] + b_r[...]
        ean_r[...] = ean
        h = _dot(gt_r[...], a_r[...]) + _dot(ean, br_r[...]) + b1_r[...]
        msg_r[...] = _dot(_lrelu(h), w2_r[...]) + b2_r[...]

    return pl.pallas_call(
        body, grid=(_GE,),
        in_specs=[_rows(_BE, _D)] * 2 + [_full((1, _D))] * 4 +
                 [_full(w1a.shape), _full(w1b.shape), _full(b1.shape),
                  _full(w2.shape), _full(b2.shape)],
        out_specs=[_rows(_BE, _D), _rows(_BE, 256)],
        out_shape=[jax.ShapeDtypeStruct((_E, _D), f32),
                   jax.ShapeDtypeStruct((_E, 256), f32)],
    )(h2, gt, ss, sq, g, b, w1a, w1b, b1, w2, b2)


def _tc_moment_fin(ssum, ssq, cnt):
    def body(s_r, q_r, c_r, mean_r, std_r):
        cm = jnp.maximum(c_r[:, 0:1], 1.0)
        mean = s_r[...] / cm
        var = _lrelu(q_r[...] / cm - mean * mean)
        mean_r[...] = mean
        std_r[...] = jnp.sqrt(var + 1e-6)

    return pl.pallas_call(
        body, grid=(_GN,),
        in_specs=[_rows(_BN, 256), _rows(_BN, 256), _rows(_BN, 128)],
        out_specs=[_rows(_BN, 256), _rows(_BN, 256)],
        out_shape=[jax.ShapeDtypeStruct((_N, 256), f32),
                   jax.ShapeDtypeStruct((_N, 256), f32)],
    )(ssum, ssq, cnt)


def _tc_src_update(xs, mean, std, s3, s4, cnt, xu, ws, b1, w2, b2):
    def body(xs_r, me_r, st_r, s3_r, s4_r, c_r, xu_r,
             w0, w1, w2_, w3, w4, w5, b1_r, wo, b2_r, h_r, ss_r, sq_r):
        cm = jnp.maximum(c_r[:, 0:1], 1.0)
        st = st_r[...]
        st2 = st * st
        skew = s3_r[...] / cm / (st2 * st)
        kurt = s4_r[...] / cm / (st2 * st2)
        h = _dot(xs_r[...], w0[...]) + _dot(me_r[...], w1[...])
        h += _dot(st, w2_[...]) + _dot(skew, w3[...]) + _dot(kurt, w4[...])
        h += _dot(xu_r[...], w5[...]) + b1_r[...]
        y = _dot(_lrelu(h), wo[...]) + b2_r[...]
        h_r[...] = y
        _acc_stats(y, ss_r, sq_r)

    return pl.pallas_call(
        body, grid=(_GN,),
        in_specs=[_rows(_BN, _D), _rows(_BN, 256), _rows(_BN, 256),
                  _rows(_BN, 256), _rows(_BN, 256), _rows(_BN, 128),
                  _full((1, _D))] +
                 [_full(w.shape) for w in ws] +
                 [_full(b1.shape), _full(w2.shape), _full(b2.shape)],
        out_specs=[_rows(_BN, _D), _full((1, _D)), _full((1, _D))],
        out_shape=[jax.ShapeDtypeStruct((_N, _D), f32),
                   jax.ShapeDtypeStruct((1, _D), f32),
                   jax.ShapeDtypeStruct((1, _D), f32)],
    )(xs, mean, std, s3, s4, cnt, xu, *ws, b1, w2, b2)


def _tc_bn_rows(h, ss, sq, g, b, nrows):
    def body(h_r, ss_r, sq_r, g_r, b_r, xn_r, cs_r):
        m = ss_r[...] * (1.0 / nrows)
        v = sq_r[...] * (1.0 / nrows) - m * m
        xn = (h_r[...] - m) * lax.rsqrt(v + 1e-5) * g_r[...] + b_r[...]
        xn_r[...] = xn

        @pl.when(pl.program_id(0) == 0)
        def _():
            cs_r[...] = jnp.zeros_like(cs_r)
        cs_r[...] += jnp.sum(xn, 0, keepdims=True)

    return pl.pallas_call(
        body, grid=(_GN,),
        in_specs=[_rows(_BN, _D)] + [_full((1, _D))] * 4,
        out_specs=[_rows(_BN, _D), _full((1, _D))],
        out_shape=[jax.ShapeDtypeStruct((_N, _D), f32),
                   jax.ShapeDtypeStruct((1, _D), f32)],
    )(h, ss, sq, g, b)


def _tc_tgt_msg(gsn, ean, w1a, w1b, b1, w2, b2):
    def body(gs_r, ea_r, a_r, br_r, b1_r, w2_r, b2_r, msg_r):
        h = _dot(gs_r[...], a_r[...]) + _dot(ea_r[...], br_r[...]) + b1_r[...]
        msg_r[...] = _dot(_lrelu(h), w2_r[...]) + b2_r[...]

    return pl.pallas_call(
        body, grid=(_GE,),
        in_specs=[_rows(_BE, _D)] * 2 + [_full(w1a.shape), _full(w1b.shape),
                  _full(b1.shape), _full(w2.shape), _full(b2.shape)],
        out_specs=[_rows(_BE, 256)],
        out_shape=[jax.ShapeDtypeStruct((_E, 256), f32)],
    )(gsn, ean, w1a, w1b, b1, w2, b2)


def _tc_tgt_update(xt, agg, xu, w1a, w1b, w1c, b1, w2, b2):
    def body(xt_r, ag_r, xu_r, a_r, br_r, c_r, b1_r, w2_r, b2_r,
             h_r, ss_r, sq_r):
        h = _dot(xt_r[...], a_r[...]) + _dot(ag_r[...], br_r[...])
        h += _dot(xu_r[...], c_r[...]) + b1_r[...]
        y = _dot(_lrelu(h), w2_r[...]) + b2_r[...]
        h_r[...] = y
        _acc_stats(y, ss_r, sq_r)

    return pl.pallas_call(
        body, grid=(_GN,),
        in_specs=[_rows(_BN, _D), _rows(_BN, 256), _full((1, _D)),
                  _full(w1a.shape), _full(w1b.shape), _full(w1c.shape),
                  _full(b1.shape), _full(w2.shape), _full(b2.shape)],
        out_specs=[_rows(_BN, _D), _full((1, _D)), _full((1, _D))],
        out_shape=[jax.ShapeDtypeStruct((_N, _D), f32),
                   jax.ShapeDtypeStruct((1, _D), f32),
                   jax.ShapeDtypeStruct((1, _D), f32)],
    )(xt, agg, xu, w1a, w1b, w1c, b1, w2, b2)


def _tc_global(xu, cs, ct, w1a, w1b, w1c, b1, w2, b2, rg):
    def body(xu_r, cs_r, ct_r, a_r, br_r, c_r, b1_r, w2_r, b2_r, rg_r, o_r):
        ms = cs_r[...] * (1.0 / _N)
        mt = ct_r[...] * (1.0 / _N)
        h = _dot(xu_r[...], a_r[...]) + _dot(ms, br_r[...])
        h += _dot(mt, c_r[...]) + b1_r[...]
        y = _dot(_lrelu(h), w2_r[...]) + b2_r[...]
        den = lax.rsqrt(jnp.mean(y * y, axis=-1, keepdims=True)
                        + jnp.finfo(jnp.float32).eps)
        o_r[...] = y * den * rg_r[...]

    return pl.pallas_call(
        body, grid=(1,),
        in_specs=[_full((1, _D))] * 3 + [_full(w1a.shape), _full(w1b.shape),
                  _full(w1c.shape), _full(b1.shape), _full(w2.shape),
                  _full(b2.shape), _full((1, _D))],
        out_specs=[_full((1, _D))],
        out_shape=[jax.ShapeDtypeStruct((1, _D), f32)],
    )(xu, cs, ct, w1a, w1b, w1c, b1, w2, b2, rg)


def _sc_gather_pair(x_s, x_t, src, tgt):
    return _build_gather(2)(x_s, x_t, src, tgt)


def _sc_gather_one(tab, idx):
    return _build_gather(1)(tab, idx)[0]


def _sc_stats(msg, src_pad):
    return _build_stats()(msg, src_pad)


def _sc_mom2(msg, src_pad, mean_a, mean_b):
    return _build_mom2()(msg, src_pad, mean_a, mean_b)


def _sc_agg(msg_t, tgt_pad):
    return _build_agg()(msg_t, tgt_pad)[0]


def _pad_idx(v):
    r = v.reshape(_NS, _NCHS, _CE)
    r = jnp.pad(r, ((0, 0), (0, _IPAD - _NCHS), (0, 0)))
    return r.reshape(_NS * _IPAD, _CE)


def kernel(x_s, x_t, edge_index, edge_attr, x_u, params):
    p = params
    src = edge_index[0].astype(jnp.int32)
    tgt = edge_index[1].astype(jnp.int32)
    src_pad = _pad_idx(src)
    tgt_pad = _pad_idx(tgt)
    r2 = lambda a: a.reshape(1, -1)

    gs, gt = _sc_gather_pair(x_s, x_t, src, tgt)

    we1 = p['We1']
    h2, es, eq = _tc_edge_mlp(gs, gt, edge_attr, x_u,
                              we1[0:128], we1[128:256], we1[256:384],
                              we1[384:512], r2(p['be1']), p['We2'],
                              r2(p['be2']))

    wsm1 = p['Wsm1']
    ean, msg = _tc_bn_edge_msg(h2, gt, es, eq, r2(p['bne_g']), r2(p['bne_b']),
                               wsm1[:128], wsm1[128:], r2(p['bsm1']),
                               p['Wsm2'], r2(p['bsm2']))

    ssum, ssq, cnt = _sc_stats(msg, src_pad)
    mean, std = _tc_moment_fin(ssum, ssq, cnt)
    s3, s4 = _sc_mom2(msg, src_pad, mean[:, 0:128], mean[:, 128:256])

    wsu1 = p['Wsu1']
    ws = (wsu1[0:128], wsu1[128:384], wsu1[384:640], wsu1[640:896],
          wsu1[896:1152], wsu1[1152:1280])
    h_s, fs, fq = _tc_src_update(x_s, mean, std, s3, s4, cnt, x_u,
                                 ws, r2(p['bsu1']), p['Wsu2'], r2(p['bsu2']))
    x_s_new, cs_sum = _tc_bn_rows(h_s, fs, fq, r2(p['bns_g']), r2(p['bns_b']),
                                  _N)

    gsn = _sc_gather_one(x_s_new, src)
    wtm1 = p['Wtm1']
    msg_t = _tc_tgt_msg(gsn, ean, wtm1[:128], wtm1[128:], r2(p['btm1']),
                        p['Wtm2'], r2(p['btm2']))[0]
    agg = _sc_agg(msg_t, tgt_pad)

    wtu1 = p['Wtu1']
    h_t, ts, tq = _tc_tgt_update(x_t, agg, x_u, wtu1[0:128], wtu1[128:384],
                                 wtu1[384:512], r2(p['btu1']), p['Wtu2'],
                                 r2(p['btu2']))
    x_t_new, ct_sum = _tc_bn_rows(h_t, ts, tq, r2(p['bnt_g']), r2(p['bnt_b']),
                                  _N)

    wg1 = p['Wg1']
    x_u_new = _tc_global(x_u, cs_sum, ct_sum, wg1[0:128], wg1[128:256],
                         wg1[256:384], r2(p['bg1']), p['Wg2'], r2(p['bg2']),
                         r2(p['rms_g']))[0]

    return (x_s_new, x_t_new, edge_index, ean, x_u_new)


# speedup vs baseline: 2.5646x; 1.1652x over previous
import functools

import jax
import jax.numpy as jnp
from jax import lax
from jax.experimental import pallas as pl
from jax.experimental.pallas import tpu as pltpu
from jax.experimental.pallas import tpu_sc as plsc

_N = 10000
_E = 160000
_D = 128
_SLOPE = 0.01
_NC, _NS, _L = 2, 16, 16
_NW = _NC * _NS

_CE = 80
_EPT = _E // _NS
_NCHS = _EPT // _CE
_IPAD = 128
_CZ = 80
_NZ = _N // _CZ

f32 = jnp.float32


def _lrelu(x):
    return jnp.where(x >= 0, x, _SLOPE * x)


def _mesh():
    return plsc.VectorSubcoreMesh(core_axis_name="c", subcore_axis_name="s")


def _fill(buf, nrows, ncols, val):
    def row(r, _):
        for g in range(ncols // _L):
            buf[r, pl.ds(g * _L, _L)] = jnp.full((_L,), val, f32)
        return 0
    lax.fori_loop(0, nrows, row, 0)


def _rr_chunks(sid, fn):
    for k in range(-(-_NZ // _NS)):
        j = sid + _NS * k
        pl.when(j < _NZ)(functools.partial(fn, j))


def _build_gather(n_tab):
    ew = _E // _NW
    c = 200
    nch = ew // c

    def body(*refs):
        tabs = refs[:n_tab]
        idxs = refs[n_tab:2 * n_tab]
        outs = refs[2 * n_tab:3 * n_tab]
        idx_v, rows_v, sem = refs[3 * n_tab:]
        wid = lax.axis_index("s") * _NC + lax.axis_index("c")
        base = wid * ew
        for tab, ih, oh in zip(tabs, idxs, outs):
            pltpu.sync_copy(ih.at[pl.ds(base, ew)], idx_v)

            def step(k, _):
                pltpu.async_copy(tab.at[idx_v.at[pl.ds(k * c, c)]], rows_v,
                                 sem).wait()
                pltpu.sync_copy(rows_v, oh.at[pl.ds(base + k * c, c)])
                return 0
            lax.fori_loop(0, nch, step, 0)

    return pl.kernel(
        body,
        out_type=[jax.ShapeDtypeStruct((_E, _D), f32)] * n_tab,
        mesh=_mesh(),
        scratch_types=[
            pltpu.VMEM((ew,), jnp.int32),
            pltpu.VMEM((c, _D), f32),
            pltpu.SemaphoreType.DMA,
        ],
    )


def _stats_body(msg_h, srcp_h, s1_h, s2_h, s3_h, s4_h, cnt_h,
                idx_v, mbuf, sbuf, acc):
    cid = lax.axis_index("c")
    sid = lax.axis_index("s")
    pltpu.sync_copy(srcp_h.at[pl.ds(sid * _IPAD, _IPAD)], idx_v)

    def one_pass(f0, kind):
        _fill(mbuf, _CZ, 128, 0.0)

        def zero(j):
            pltpu.sync_copy(mbuf, acc.at[pl.ds(j * _CZ, _CZ)])
        _rr_chunks(sid, zero)
        if kind == "cnt":
            _fill(sbuf, _CE, 128, 1.0)
        plsc.subcore_barrier()

        def step(j, _):
            idxr = idx_v.at[j]
            if kind == "cnt":
                pltpu.sync_copy(sbuf, acc.at[idxr], add=True)
                return 0
            e0 = sid * _EPT + j * _CE
            pltpu.sync_copy(msg_h.at[pl.ds(e0, _CE), pl.ds(f0, 128)], mbuf)
            if kind == "sum":
                pltpu.sync_copy(mbuf, acc.at[idxr], add=True)
            else:
                def powrow(r, _):
                    for g in range(128 // _L):
                        v = mbuf[r, pl.ds(g * _L, _L)]
                        v2 = v * v
                        if kind == "sq":
                            sbuf[r, pl.ds(g * _L, _L)] = v2
                        elif kind == "cube":
                            sbuf[r, pl.ds(g * _L, _L)] = v2 * v
                        else:
                            sbuf[r, pl.ds(g * _L, _L)] = v2 * v2
                    return 0
                lax.fori_loop(0, _CE, powrow, 0)
                pltpu.sync_copy(sbuf, acc.at[idxr], add=True)
            return 0
        lax.fori_loop(0, _NCHS, step, 0)
        plsc.subcore_barrier()

        out_h = {"sum": s1_h, "sq": s2_h, "cube": s3_h, "quart": s4_h,
                 "cnt": cnt_h}[kind]

        def read(j):
            r0 = j * _CZ
            pltpu.sync_copy(acc.at[pl.ds(r0, _CZ)], mbuf)
            if kind == "cnt":
                pltpu.sync_copy(mbuf, out_h.at[pl.ds(r0, _CZ)])
            else:
                pltpu.sync_copy(mbuf,
                                out_h.at[pl.ds(r0, _CZ), pl.ds(f0, 128)])
        _rr_chunks(sid, read)
        plsc.subcore_barrier()

    for f0, core, kind in ((0, 0, "sum"), (128, 0, "sum"),
                           (0, 0, "cube"), (128, 0, "cube"), (0, 0, "cnt"),
                           (0, 1, "sq"), (128, 1, "sq"),
                           (0, 1, "quart"), (128, 1, "quart")):
        pl.when(cid == core)(functools.partial(one_pass, f0, kind))


def _build_stats():
    return pl.kernel(
        _stats_body,
        out_type=[jax.ShapeDtypeStruct((_N, 256), f32),
                  jax.ShapeDtypeStruct((_N, 256), f32),
                  jax.ShapeDtypeStruct((_N, 256), f32),
                  jax.ShapeDtypeStruct((_N, 256), f32),
                  jax.ShapeDtypeStruct((_N, 128), f32)],
        mesh=_mesh(),
        scratch_types=[
            pltpu.VMEM((_IPAD, _CE), jnp.int32),
            pltpu.VMEM((_CE, 128), f32),
            pltpu.VMEM((_CE, 128), f32),
            pltpu.VMEM_SHARED((_N, 128), f32),
        ],
    )


def _agg_body(msg_h, tgtp_h, agg_h, idx_v, mbuf, acc):
    cid = lax.axis_index("c")
    sid = lax.axis_index("s")
    pltpu.sync_copy(tgtp_h.at[pl.ds(sid * _IPAD, _IPAD)], idx_v)

    def one_pass(f0):
        _fill(mbuf, _CZ, 128, 0.0)

        def zero(j):
            pltpu.sync_copy(mbuf, acc.at[pl.ds(j * _CZ, _CZ)])
        _rr_chunks(sid, zero)
        plsc.subcore_barrier()

        def step(j, _):
            e0 = sid * _EPT + j * _CE
            pltpu.sync_copy(msg_h.at[pl.ds(e0, _CE), pl.ds(f0, 128)], mbuf)
            pltpu.sync_copy(mbuf, acc.at[idx_v.at[j]], add=True)
            return 0
        lax.fori_loop(0, _NCHS, step, 0)
        plsc.subcore_barrier()

        def read(j):
            r0 = j * _CZ
            pltpu.sync_copy(acc.at[pl.ds(r0, _CZ)], mbuf)
            pltpu.sync_copy(mbuf, agg_h.at[pl.ds(r0, _CZ), pl.ds(f0, 128)])
        _rr_chunks(sid, read)
        plsc.subcore_barrier()

    pl.when(cid == 0)(functools.partial(one_pass, 0))
    pl.when(cid == 1)(functools.partial(one_pass, 128))


def _build_agg():
    return pl.kernel(
        _agg_body,
        out_type=[jax.ShapeDtypeStruct((_N, 256), f32)],
        mesh=_mesh(),
        scratch_types=[
            pltpu.VMEM((_IPAD, _CE), jnp.int32),
            pltpu.VMEM((_CE, 128), f32),
            pltpu.VMEM_SHARED((_N, 128), f32),
        ],
    )


_BE = 2000
_GE = _E // _BE
_BN = 1000
_GN = _N // _BN


def _full(shape):
    return pl.BlockSpec(shape, lambda i: (0,) * len(shape))


def _rows(b, w):
    return pl.BlockSpec((b, w), lambda i: (i, 0))


def _dot(a, b):
    return jnp.dot(a, b, preferred_element_type=f32)


def _acc_stats(y, ss_r, sq_r):
    @pl.when(pl.program_id(0) == 0)
    def _():
        ss_r[...] = jnp.zeros_like(ss_r)
        sq_r[...] = jnp.zeros_like(sq_r)
    ss_r[...] += jnp.sum(y, 0, keepdims=True)
    sq_r[...] += jnp.sum(y * y, 0, keepdims=True)


def _tc_edge_mlp(gs, gt, ea, xu, w1a, w1b, w1c, w1d, b1, w2, b2):
    def body(gs_r, gt_r, ea_r, xu_r, a_r, br_r, c_r, d_r, b1_r, w2_r, b2_r,
             h2_r, ss_r, sq_r):
        h = _dot(gs_r[...], a_r[...]) + _dot(gt_r[...], br_r[...])
        h += _dot(ea_r[...], c_r[...])
        h += _dot(xu_r[...], d_r[...]) + b1_r[...]
        y = _dot(_lrelu(h), w2_r[...]) + b2_r[...]
        h2_r[...] = y
        _acc_stats(y, ss_r, sq_r)

    return pl.pallas_call(
        body, grid=(_GE,),
        in_specs=[_rows(_BE, _D)] * 3 + [_full(xu.shape), _full(w1a.shape),
                  _full(w1b.shape), _full(w1c.shape), _full(w1d.shape),
                  _full(b1.shape), _full(w2.shape), _full(b2.shape)],
        out_specs=[_rows(_BE, _D), _full((1, _D)), _full((1, _D))],
        out_shape=[jax.ShapeDtypeStruct((_E, _D), f32),
                   jax.ShapeDtypeStruct((1, _D), f32),
                   jax.ShapeDtypeStruct((1, _D), f32)],
    )(gs, gt, ea, xu, w1a, w1b, w1c, w1d, b1, w2, b2)


def _tc_bn_edge_msg(h2, gt, ss, sq, g, b, w1a, w1b, b1, w2, b2):
    def body(h2_r, gt_r, ss_r, sq_r, g_r, b_r, a_r, br_r, b1_r, w2_r, b2_r,
             ean_r, msg_r):
        m = ss_r[...] * (1.0 / _E)
        v = sq_r[...] * (1.0 / _E) - m * m
        inv = lax.rsqrt(v + 1e-5)
        ean = (h2_r[...] - m) * inv * g_r[...] + b_r[...]
        ean_r[...] = ean
        h = _dot(gt_r[...], a_r[...]) + _dot(ean, br_r[...]) + b1_r[...]
        msg_r[...] = _dot(_lrelu(h), w2_r[...]) + b2_r[...]

    return pl.pallas_call(
        body, grid=(_GE,),
        in_specs=[_rows(_BE, _D)] * 2 + [_full((1, _D))] * 4 +
                 [_full(w1a.shape), _full(w1b.shape), _full(b1.shape),
                  _full(w2.shape), _full(b2.shape)],
        out_specs=[_rows(_BE, _D), _rows(_BE, 256)],
        out_shape=[jax.ShapeDtypeStruct((_E, _D), f32),
                   jax.ShapeDtypeStruct((_E, 256), f32)],
    )(h2, gt, ss, sq, g, b, w1a, w1b, b1, w2, b2)


def _tc_moment_fin(s1, s2, s3, s4, cnt):
    def body(s1_r, s2_r, s3_r, s4_r, c_r, mean_r, std_r, skew_r, kurt_r):
        cm = jnp.maximum(c_r[:, 0:1], 1.0)
        m1 = s1_r[...] / cm
        m2 = s2_r[...] / cm
        m3 = s3_r[...] / cm
        m4 = s4_r[...] / cm
        var = _lrelu(m2 - m1 * m1)
        std = jnp.sqrt(var + 1e-6)
        m1sq = m1 * m1
        c3 = m3 - 3.0 * m1 * m2 + 2.0 * m1sq * m1
        c4 = m4 - 4.0 * m1 * m3 + 6.0 * m1sq * m2 - 3.0 * m1sq * m1sq
        std2 = std * std
        mean_r[...] = m1
        std_r[...] = std
        skew_r[...] = c3 / (std2 * std)
        kurt_r[...] = c4 / (std2 * std2)

    return pl.pallas_call(
        body, grid=(_GN,),
        in_specs=[_rows(_BN, 256)] * 4 + [_rows(_BN, 128)],
        out_specs=[_rows(_BN, 256)] * 4,
        out_shape=[jax.ShapeDtypeStruct((_N, 256), f32)] * 4,
    )(s1, s2, s3, s4, cnt)


def _tc_src_update(xs, mean, std, skew, kurt, xu, ws, b1, w2, b2):
    def body(xs_r, me_r, st_r, sk_r, ku_r, xu_r,
             w0, w1, w2_, w3, w4, w5, b1_r, wo, b2_r, h_r, ss_r, sq_r):
        h = _dot(xs_r[...], w0[...]) + _dot(me_r[...], w1[...])
        h += _dot(st_r[...], w2_[...]) + _dot(sk_r[...], w3[...])
        h += _dot(ku_r[...], w4[...])
        h += _dot(xu_r[...], w5[...]) + b1_r[...]
        y = _dot(_lrelu(h), wo[...]) + b2_r[...]
        h_r[...] = y
        _acc_stats(y, ss_r, sq_r)

    return pl.pallas_call(
        body, grid=(_GN,),
        in_specs=[_rows(_BN, _D), _rows(_BN, 256), _rows(_BN, 256),
                  _rows(_BN, 256), _rows(_BN, 256), _full((1, _D))] +
                 [_full(w.shape) for w in ws] +
                 [_full(b1.shape), _full(w2.shape), _full(b2.shape)],
        out_specs=[_rows(_BN, _D), _full((1, _D)), _full((1, _D))],
        out_shape=[jax.ShapeDtypeStruct((_N, _D), f32),
                   jax.ShapeDtypeStruct((1, _D), f32),
                   jax.ShapeDtypeStruct((1, _D), f32)],
    )(xs, mean, std, skew, kurt, xu, *ws, b1, w2, b2)


def _tc_bn_rows(h, ss, sq, g, b, nrows):
    def body(h_r, ss_r, sq_r, g_r, b_r, xn_r, cs_r):
        m = ss_r[...] * (1.0 / nrows)
        v = sq_r[...] * (1.0 / nrows) - m * m
        xn = (h_r[...] - m) * lax.rsqrt(v + 1e-5) * g_r[...] + b_r[...]
        xn_r[...] = xn

        @pl.when(pl.program_id(0) == 0)
        def _():
            cs_r[...] = jnp.zeros_like(cs_r)
        cs_r[...] += jnp.sum(xn, 0, keepdims=True)

    return pl.pallas_call(
        body, grid=(_GN,),
        in_specs=[_rows(_BN, _D)] + [_full((1, _D))] * 4,
        out_specs=[_rows(_BN, _D), _full((1, _D))],
        out_shape=[jax.ShapeDtypeStruct((_N, _D), f32),
                   jax.ShapeDtypeStruct((1, _D), f32)],
    )(h, ss, sq, g, b)


def _tc_tgt_msg(gsn, ean, w1a, w1b, b1, w2, b2):
    def body(gs_r, ea_r, a_r, br_r, b1_r, w2_r, b2_r, msg_r):
        h = _dot(gs_r[...], a_r[...]) + _dot(ea_r[...], br_r[...]) + b1_r[...]
        msg_r[...] = _dot(_lrelu(h), w2_r[...]) + b2_r[...]

    return pl.pallas_call(
        body, grid=(_GE,),
        in_specs=[_rows(_BE, _D)] * 2 + [_full(w1a.shape), _full(w1b.shape),
                  _full(b1.shape), _full(w2.shape), _full(b2.shape)],
        out_specs=[_rows(_BE, 256)],
        out_shape=[jax.ShapeDtypeStruct((_E, 256), f32)],
    )(gsn, ean, w1a, w1b, b1, w2, b2)


def _tc_tgt_update(xt, agg, xu, w1a, w1b, w1c, b1, w2, b2):
    def body(xt_r, ag_r, xu_r, a_r, br_r, c_r, b1_r, w2_r, b2_r,
             h_r, ss_r, sq_r):
        h = _dot(xt_r[...], a_r[...]) + _dot(ag_r[...], br_r[...])
        h += _dot(xu_r[...], c_r[...]) + b1_r[...]
        y = _dot(_lrelu(h), w2_r[...]) + b2_r[...]
        h_r[...] = y
        _acc_stats(y, ss_r, sq_r)

    return pl.pallas_call(
        body, grid=(_GN,),
        in_specs=[_rows(_BN, _D), _rows(_BN, 256), _full((1, _D)),
                  _full(w1a.shape), _full(w1b.shape), _full(w1c.shape),
                  _full(b1.shape), _full(w2.shape), _full(b2.shape)],
        out_specs=[_rows(_BN, _D), _full((1, _D)), _full((1, _D))],
        out_shape=[jax.ShapeDtypeStruct((_N, _D), f32),
                   jax.ShapeDtypeStruct((1, _D), f32),
                   jax.ShapeDtypeStruct((1, _D), f32)],
    )(xt, agg, xu, w1a, w1b, w1c, b1, w2, b2)


def _tc_global(xu, cs, ct, w1a, w1b, w1c, b1, w2, b2, rg):
    def body(xu_r, cs_r, ct_r, a_r, br_r, c_r, b1_r, w2_r, b2_r, rg_r, o_r):
        ms = cs_r[...] * (1.0 / _N)
        mt = ct_r[...] * (1.0 / _N)
        h = _dot(xu_r[...], a_r[...]) + _dot(ms, br_r[...])
        h += _dot(mt, c_r[...]) + b1_r[...]
        y = _dot(_lrelu(h), w2_r[...]) + b2_r[...]
        den = lax.rsqrt(jnp.mean(y * y, axis=-1, keepdims=True)
                        + jnp.finfo(jnp.float32).eps)
        o_r[...] = y * den * rg_r[...]

    return pl.pallas_call(
        body, grid=(1,),
        in_specs=[_full((1, _D))] * 3 + [_full(w1a.shape), _full(w1b.shape),
                  _full(w1c.shape), _full(b1.shape), _full(w2.shape),
                  _full(b2.shape), _full((1, _D))],
        out_specs=[_full((1, _D))],
        out_shape=[jax.ShapeDtypeStruct((1, _D), f32)],
    )(xu, cs, ct, w1a, w1b, w1c, b1, w2, b2, rg)


def _sc_gather_pair(x_s, x_t, src, tgt):
    return _build_gather(2)(x_s, x_t, src, tgt)


def _sc_gather_one(tab, idx):
    return _build_gather(1)(tab, idx)[0]


def _sc_stats(msg, src_pad):
    return _build_stats()(msg, src_pad)


def _sc_agg(msg_t, tgt_pad):
    return _build_agg()(msg_t, tgt_pad)[0]


def _pad_idx(v):
    r = v.reshape(_NS, _NCHS, _CE)
    r = jnp.pad(r, ((0, 0), (0, _IPAD - _NCHS), (0, 0)))
    return r.reshape(_NS * _IPAD, _CE)


def kernel(x_s, x_t, edge_index, edge_attr, x_u, params):
    p = params
    src = edge_index[0].astype(jnp.int32)
    tgt = edge_index[1].astype(jnp.int32)
    src_pad = _pad_idx(src)
    tgt_pad = _pad_idx(tgt)
    r2 = lambda a: a.reshape(1, -1)

    gs, gt = _sc_gather_pair(x_s, x_t, src, tgt)

    we1 = p['We1']
    h2, es, eq = _tc_edge_mlp(gs, gt, edge_attr, x_u,
                              we1[0:128], we1[128:256], we1[256:384],
                              we1[384:512], r2(p['be1']), p['We2'],
                              r2(p['be2']))

    wsm1 = p['Wsm1']
    ean, msg = _tc_bn_edge_msg(h2, gt, es, eq, r2(p['bne_g']), r2(p['bne_b']),
                               wsm1[:128], wsm1[128:], r2(p['bsm1']),
                               p['Wsm2'], r2(p['bsm2']))

    s1, s2, s3, s4, cnt = _sc_stats(msg, src_pad)
    mean, std, skew, kurt = _tc_moment_fin(s1, s2, s3, s4, cnt)

    wsu1 = p['Wsu1']
    ws = (wsu1[0:128], wsu1[128:384], wsu1[384:640], wsu1[640:896],
          wsu1[896:1152], wsu1[1152:1280])
    h_s, fs, fq = _tc_src_update(x_s, mean, std, skew, kurt, x_u,
                                 ws, r2(p['bsu1']), p['Wsu2'], r2(p['bsu2']))
    x_s_new, cs_sum = _tc_bn_rows(h_s, fs, fq, r2(p['bns_g']), r2(p['bns_b']),
                                  _N)

    gsn = _sc_gather_one(x_s_new, src)
    wtm1 = p['Wtm1']
    msg_t = _tc_tgt_msg(gsn, ean, wtm1[:128], wtm1[128:], r2(p['btm1']),
                        p['Wtm2'], r2(p['btm2']))[0]
    agg = _sc_agg(msg_t, tgt_pad)

    wtu1 = p['Wtu1']
    h_t, ts, tq = _tc_tgt_update(x_t, agg, x_u, wtu1[0:128], wtu1[128:384],
                                 wtu1[384:512], r2(p['btu1']), p['Wtu2'],
                                 r2(p['btu2']))
    x_t_new, ct_sum = _tc_bn_rows(h_t, ts, tq, r2(p['bnt_g']), r2(p['bnt_b']),
                                  _N)

    wg1 = p['Wg1']
    x_u_new = _tc_global(x_u, cs_sum, ct_sum, wg1[0:128], wg1[128:256],
                         wg1[256:384], r2(p['bg1']), p['Wg2'], r2(p['bg2']),
                         r2(p['rms_g']))[0]

    return (x_s_new, x_t_new, edge_index, ean, x_u_new)


# speedup vs baseline: 3.4283x; 1.3368x over previous
import functools

import jax
import jax.numpy as jnp
from jax import lax
from jax.experimental import pallas as pl
from jax.experimental.pallas import tpu as pltpu
from jax.experimental.pallas import tpu_sc as plsc

_N = 10000
_E = 160000
_D = 128
_SLOPE = 0.01
_NC, _NS, _L = 2, 16, 16
_NW = _NC * _NS

_CE = 80
_EPT = _E // _NS
_NCHS = _EPT // _CE
_IPAD = 128
_CZ = 80
_NZ = _N // _CZ

f32 = jnp.float32


def _lrelu(x):
    return jnp.where(x >= 0, x, _SLOPE * x)


def _mesh():
    return plsc.VectorSubcoreMesh(core_axis_name="c", subcore_axis_name="s")


def _fill(buf, nrows, ncols, val):
    def row(r, _):
        for g in range(ncols // _L):
            buf[r, pl.ds(g * _L, _L)] = jnp.full((_L,), val, f32)
        return 0
    lax.fori_loop(0, nrows, row, 0)


def _rr_chunks(sid, fn):
    for k in range(-(-_NZ // _NS)):
        j = sid + _NS * k
        pl.when(j < _NZ)(functools.partial(fn, j))


def _build_gather(n_tab):
    ew = _E // _NW
    c = 200
    nch = ew // c

    def body(*refs):
        tabs = refs[:n_tab]
        idxs = refs[n_tab:2 * n_tab]
        outs = refs[2 * n_tab:3 * n_tab]
        idx_v, r0, r1, sem0, sem1 = refs[3 * n_tab:]
        rows = (r0, r1)
        sems = (sem0, sem1)
        wid = lax.axis_index("s") * _NC + lax.axis_index("c")
        base = wid * ew
        for tab, ih, oh in zip(tabs, idxs, outs):
            pltpu.sync_copy(ih.at[pl.ds(base, ew)], idx_v)

            def g_at(k, b):
                pltpu.async_copy(tab.at[idx_v.at[pl.ds(k * c, c)]],
                                 rows[b], sems[b])

            def g_wait(b):
                pltpu.make_async_copy(tab.at[idx_v.at[pl.ds(0, c)]],
                                      rows[b], sems[b]).wait()

            g_at(0, 0)
            g_at(1, 1)

            def pair(k2, _):
                for b in range(2):
                    k = 2 * k2 + b
                    g_wait(b)
                    pltpu.sync_copy(rows[b], oh.at[pl.ds(base + k * c, c)])

                    @pl.when(k + 2 < nch)
                    def _():
                        g_at(k + 2, b)
                return 0
            lax.fori_loop(0, nch // 2, pair, 0)
            k = nch - 1
            g_wait(k % 2)
            pltpu.sync_copy(rows[k % 2], oh.at[pl.ds(base + k * c, c)])

    return pl.kernel(
        body,
        out_type=[jax.ShapeDtypeStruct((_E, _D), f32)] * n_tab,
        mesh=_mesh(),
        scratch_types=[
            pltpu.VMEM((ew,), jnp.int32),
            pltpu.VMEM((c, _D), f32),
            pltpu.VMEM((c, _D), f32),
            pltpu.SemaphoreType.DMA,
            pltpu.SemaphoreType.DMA,
        ],
    )


def _pow_into(kind, src, dst):
    def powrow(r, _):
        for g in range(128 // _L):
            v = src[r, pl.ds(g * _L, _L)]
            v2 = v * v
            if kind == "sq":
                dst[r, pl.ds(g * _L, _L)] = v2
            elif kind == "cube":
                dst[r, pl.ds(g * _L, _L)] = v2 * v
            else:
                dst[r, pl.ds(g * _L, _L)] = v2 * v2
        return 0
    lax.fori_loop(0, _CE, powrow, 0)


def _stats_body(msg_h, srcp_h, s1_h, s2_h, s3_h, s4_h, cnt_h,
                idx_v, m0, m1, sbuf, sem0, sem1, acc):
    cid = lax.axis_index("c")
    sid = lax.axis_index("s")
    mb = (m0, m1)
    sems = (sem0, sem1)
    pltpu.sync_copy(srcp_h.at[pl.ds(sid * _IPAD, _IPAD)], idx_v)

    def one_pass(f0, kind):
        _fill(m0, _CZ, 128, 0.0)

        def zero(j):
            pltpu.sync_copy(m0, acc.at[pl.ds(j * _CZ, _CZ)])
        _rr_chunks(sid, zero)
        if kind == "cnt":
            _fill(sbuf, _CE, 128, 1.0)
        plsc.subcore_barrier()

        def ld_at(j, b):
            e0 = sid * _EPT + j * _CE
            pltpu.async_copy(msg_h.at[pl.ds(e0, _CE), pl.ds(f0, 128)],
                             mb[b], sems[b])

        def ld_wait(b):
            pltpu.make_async_copy(
                msg_h.at[pl.ds(0, _CE), pl.ds(f0, 128)], mb[b],
                sems[b]).wait()

        if kind == "cnt":
            def step(j, _):
                pltpu.sync_copy(sbuf, acc.at[idx_v.at[j]], add=True)
                return 0
            lax.fori_loop(0, _NCHS, step, 0)
        else:
            ld_at(0, 0)
            ld_at(1, 1)

            def pair(k, _):
                for b in range(2):
                    j = 2 * k + b
                    ld_wait(b)
                    if kind == "sum":
                        pltpu.sync_copy(mb[b], acc.at[idx_v.at[j]], add=True)
                    else:
                        _pow_into(kind, mb[b], sbuf)
                        pltpu.sync_copy(sbuf, acc.at[idx_v.at[j]], add=True)

                    @pl.when(j + 2 < _NCHS)
                    def _():
                        ld_at(j + 2, b)
                return 0
            lax.fori_loop(0, _NCHS // 2, pair, 0)
            j = _NCHS - 1
            ld_wait(j % 2)
            if kind == "sum":
                pltpu.sync_copy(mb[j % 2], acc.at[idx_v.at[j]], add=True)
            else:
                _pow_into(kind, mb[j % 2], sbuf)
                pltpu.sync_copy(sbuf, acc.at[idx_v.at[j]], add=True)
        plsc.subcore_barrier()

        out_h = {"sum": s1_h, "sq": s2_h, "cube": s3_h, "quart": s4_h,
                 "cnt": cnt_h}[kind]

        def read(j):
            r0 = j * _CZ
            pltpu.sync_copy(acc.at[pl.ds(r0, _CZ)], m0)
            if kind == "cnt":
                pltpu.sync_copy(m0, out_h.at[pl.ds(r0, _CZ)])
            else:
                pltpu.sync_copy(m0,
                                out_h.at[pl.ds(r0, _CZ), pl.ds(f0, 128)])
        _rr_chunks(sid, read)
        plsc.subcore_barrier()

    for f0, core, kind in ((0, 0, "sum"), (128, 0, "sum"),
                           (0, 0, "cube"), (128, 0, "cube"), (0, 0, "cnt"),
                           (0, 1, "sq"), (128, 1, "sq"),
                           (0, 1, "quart"), (128, 1, "quart")):
        pl.when(cid == core)(functools.partial(one_pass, f0, kind))


def _build_stats():
    return pl.kernel(
        _stats_body,
        out_type=[jax.ShapeDtypeStruct((_N, 256), f32),
                  jax.ShapeDtypeStruct((_N, 256), f32),
                  jax.ShapeDtypeStruct((_N, 256), f32),
                  jax.ShapeDtypeStruct((_N, 256), f32),
                  jax.ShapeDtypeStruct((_N, 128), f32)],
        mesh=_mesh(),
        scratch_types=[
            pltpu.VMEM((_IPAD, _CE), jnp.int32),
            pltpu.VMEM((_CE, 128), f32),
            pltpu.VMEM((_CE, 128), f32),
            pltpu.VMEM((_CE, 128), f32),
            pltpu.SemaphoreType.DMA,
            pltpu.SemaphoreType.DMA,
            pltpu.VMEM_SHARED((_N, 128), f32),
        ],
    )


def _agg_body(msg_h, tgtp_h, agg_h, idx_v, m0, m1, sem0, sem1, acc):
    cid = lax.axis_index("c")
    sid = lax.axis_index("s")
    mb = (m0, m1)
    sems = (sem0, sem1)
    pltpu.sync_copy(tgtp_h.at[pl.ds(sid * _IPAD, _IPAD)], idx_v)

    def one_pass(f0):
        _fill(m0, _CZ, 128, 0.0)

        def zero(j):
            pltpu.sync_copy(m0, acc.at[pl.ds(j * _CZ, _CZ)])
        _rr_chunks(sid, zero)
        plsc.subcore_barrier()

        def ld_at(j, b):
            e0 = sid * _EPT + j * _CE
            pltpu.async_copy(msg_h.at[pl.ds(e0, _CE), pl.ds(f0, 128)],
                             mb[b], sems[b])

        def ld_wait(b):
            pltpu.make_async_copy(
                msg_h.at[pl.ds(0, _CE), pl.ds(f0, 128)], mb[b],
                sems[b]).wait()

        ld_at(0, 0)
        ld_at(1, 1)

        def pair(k, _):
            for b in range(2):
                j = 2 * k + b
                ld_wait(b)
                pltpu.sync_copy(mb[b], acc.at[idx_v.at[j]], add=True)

                @pl.when(j + 2 < _NCHS)
                def _():
                    ld_at(j + 2, b)
            return 0
        lax.fori_loop(0, _NCHS // 2, pair, 0)
        j = _NCHS - 1
        ld_wait(j % 2)
        pltpu.sync_copy(mb[j % 2], acc.at[idx_v.at[j]], add=True)
        plsc.subcore_barrier()

        def read(j):
            r0 = j * _CZ
            pltpu.sync_copy(acc.at[pl.ds(r0, _CZ)], m0)
            pltpu.sync_copy(m0, agg_h.at[pl.ds(r0, _CZ), pl.ds(f0, 128)])
        _rr_chunks(sid, read)
        plsc.subcore_barrier()

    pl.when(cid == 0)(functools.partial(one_pass, 0))
    pl.when(cid == 1)(functools.partial(one_pass, 128))


def _build_agg():
    return pl.kernel(
        _agg_body,
        out_type=[jax.ShapeDtypeStruct((_N, 256), f32)],
        mesh=_mesh(),
        scratch_types=[
            pltpu.VMEM((_IPAD, _CE), jnp.int32),
            pltpu.VMEM((_CE, 128), f32),
            pltpu.VMEM((_CE, 128), f32),
            pltpu.SemaphoreType.DMA,
            pltpu.SemaphoreType.DMA,
            pltpu.VMEM_SHARED((_N, 128), f32),
        ],
    )


_BE = 2000
_GE = _E // _BE
_BN = 1000
_GN = _N // _BN


def _full(shape):
    return pl.BlockSpec(shape, lambda i: (0,) * len(shape))


def _rows(b, w):
    return pl.BlockSpec((b, w), lambda i: (i, 0))


def _dot(a, b):
    return jnp.dot(a, b, preferred_element_type=f32)


def _acc_stats(y, ss_r, sq_r):
    @pl.when(pl.program_id(0) == 0)
    def _():
        ss_r[...] = jnp.zeros_like(ss_r)
        sq_r[...] = jnp.zeros_like(sq_r)
    ss_r[...] += jnp.sum(y, 0, keepdims=True)
    sq_r[...] += jnp.sum(y * y, 0, keepdims=True)


def _tc_edge_mlp(gs, gt, ea, xu, w1a, w1b, w1c, w1d, b1, w2, b2):
    def body(gs_r, gt_r, ea_r, xu_r, a_r, br_r, c_r, d_r, b1_r, w2_r, b2_r,
             h2_r, ss_r, sq_r):
        h = _dot(gs_r[...], a_r[...]) + _dot(gt_r[...], br_r[...])
        h += _dot(ea_r[...], c_r[...])
        h += _dot(xu_r[...], d_r[...]) + b1_r[...]
        y = _dot(_lrelu(h), w2_r[...]) + b2_r[...]
        h2_r[...] = y
        _acc_stats(y, ss_r, sq_r)

    return pl.pallas_call(
        body, grid=(_GE,),
        in_specs=[_rows(_BE, _D)] * 3 + [_full(xu.shape), _full(w1a.shape),
                  _full(w1b.shape), _full(w1c.shape), _full(w1d.shape),
                  _full(b1.shape), _full(w2.shape), _full(b2.shape)],
        out_specs=[_rows(_BE, _D), _full((1, _D)), _full((1, _D))],
        out_shape=[jax.ShapeDtypeStruct((_E, _D), f32),
                   jax.ShapeDtypeStruct((1, _D), f32),
                   jax.ShapeDtypeStruct((1, _D), f32)],
    )(gs, gt, ea, xu, w1a, w1b, w1c, w1d, b1, w2, b2)


def _tc_bn_edge_msg(h2, gt, ss, sq, g, b, w1a, w1b, b1, w2, b2):
    def body(h2_r, gt_r, ss_r, sq_r, g_r, b_r, a_r, br_r, b1_r, w2_r, b2_r,
             ean_r, msg_r):
        m = ss_r[...] * (1.0 / _E)
        v = sq_r[...] * (1.0 / _E) - m * m
        inv = lax.rsqrt(v + 1e-5)
        ean = (h2_r[...] - m) * inv * g_r[...] + b_r[...]
        ean_r[...] = ean
        h = _dot(gt_r[...], a_r[...]) + _dot(ean, br_r[...]) + b1_r[...]
        msg_r[...] = _dot(_lrelu(h), w2_r[...]) + b2_r[...]

    return pl.pallas_call(
        body, grid=(_GE,),
        in_specs=[_rows(_BE, _D)] * 2 + [_full((1, _D))] * 4 +
                 [_full(w1a.shape), _full(w1b.shape), _full(b1.shape),
                  _full(w2.shape), _full(b2.shape)],
        out_specs=[_rows(_BE, _D), _rows(_BE, 256)],
        out_shape=[jax.ShapeDtypeStruct((_E, _D), f32),
                   jax.ShapeDtypeStruct((_E, 256), f32)],
    )(h2, gt, ss, sq, g, b, w1a, w1b, b1, w2, b2)


def _tc_moment_fin(s1, s2, s3, s4, cnt):
    def body(s1_r, s2_r, s3_r, s4_r, c_r, mean_r, std_r, skew_r, kurt_r):
        cm = jnp.maximum(c_r[:, 0:1], 1.0)
        m1 = s1_r[...] / cm
        m2 = s2_r[...] / cm
        m3 = s3_r[...] / cm
        m4 = s4_r[...] / cm
        var = _lrelu(m2 - m1 * m1)
        std = jnp.sqrt(var + 1e-6)
        m1sq = m1 * m1
        c3 = m3 - 3.0 * m1 * m2 + 2.0 * m1sq * m1
        c4 = m4 - 4.0 * m1 * m3 + 6.0 * m1sq * m2 - 3.0 * m1sq * m1sq
        std2 = std * std
        mean_r[...] = m1
        std_r[...] = std
        skew_r[...] = c3 / (std2 * std)
        kurt_r[...] = c4 / (std2 * std2)

    return pl.pallas_call(
        body, grid=(_GN,),
        in_specs=[_rows(_BN, 256)] * 4 + [_rows(_BN, 128)],
        out_specs=[_rows(_BN, 256)] * 4,
        out_shape=[jax.ShapeDtypeStruct((_N, 256), f32)] * 4,
    )(s1, s2, s3, s4, cnt)


def _tc_src_update(xs, mean, std, skew, kurt, xu, ws, b1, w2, b2):
    def body(xs_r, me_r, st_r, sk_r, ku_r, xu_r,
             w0, w1, w2_, w3, w4, w5, b1_r, wo, b2_r, h_r, ss_r, sq_r):
        h = _dot(xs_r[...], w0[...]) + _dot(me_r[...], w1[...])
        h += _dot(st_r[...], w2_[...]) + _dot(sk_r[...], w3[...])
        h += _dot(ku_r[...], w4[...])
        h += _dot(xu_r[...], w5[...]) + b1_r[...]
        y = _dot(_lrelu(h), wo[...]) + b2_r[...]
        h_r[...] = y
        _acc_stats(y, ss_r, sq_r)

    return pl.pallas_call(
        body, grid=(_GN,),
        in_specs=[_rows(_BN, _D), _rows(_BN, 256), _rows(_BN, 256),
                  _rows(_BN, 256), _rows(_BN, 256), _full((1, _D))] +
                 [_full(w.shape) for w in ws] +
                 [_full(b1.shape), _full(w2.shape), _full(b2.shape)],
        out_specs=[_rows(_BN, _D), _full((1, _D)), _full((1, _D))],
        out_shape=[jax.ShapeDtypeStruct((_N, _D), f32),
                   jax.ShapeDtypeStruct((1, _D), f32),
                   jax.ShapeDtypeStruct((1, _D), f32)],
    )(xs, mean, std, skew, kurt, xu, *ws, b1, w2, b2)


def _tc_bn_rows(h, ss, sq, g, b, nrows):
    def body(h_r, ss_r, sq_r, g_r, b_r, xn_r, cs_r):
        m = ss_r[...] * (1.0 / nrows)
        v = sq_r[...] * (1.0 / nrows) - m * m
        xn = (h_r[...] - m) * lax.rsqrt(v + 1e-5) * g_r[...] + b_r[...]
        xn_r[...] = xn

        @pl.when(pl.program_id(0) == 0)
        def _():
            cs_r[...] = jnp.zeros_like(cs_r)
        cs_r[...] += jnp.sum(xn, 0, keepdims=True)

    return pl.pallas_call(
        body, grid=(_GN,),
        in_specs=[_rows(_BN, _D)] + [_full((1, _D))] * 4,
        out_specs=[_rows(_BN, _D), _full((1, _D))],
        out_shape=[jax.ShapeDtypeStruct((_N, _D), f32),
                   jax.ShapeDtypeStruct((1, _D), f32)],
    )(h, ss, sq, g, b)


def _tc_tgt_msg(gsn, ean, w1a, w1b, b1, w2, b2):
    def body(gs_r, ea_r, a_r, br_r, b1_r, w2_r, b2_r, msg_r):
        h = _dot(gs_r[...], a_r[...]) + _dot(ea_r[...], br_r[...]) + b1_r[...]
        msg_r[...] = _dot(_lrelu(h), w2_r[...]) + b2_r[...]

    return pl.pallas_call(
        body, grid=(_GE,),
        in_specs=[_rows(_BE, _D)] * 2 + [_full(w1a.shape), _full(w1b.shape),
                  _full(b1.shape), _full(w2.shape), _full(b2.shape)],
        out_specs=[_rows(_BE, 256)],
        out_shape=[jax.ShapeDtypeStruct((_E, 256), f32)],
    )(gsn, ean, w1a, w1b, b1, w2, b2)


def _tc_tgt_update(xt, agg, xu, w1a, w1b, w1c, b1, w2, b2):
    def body(xt_r, ag_r, xu_r, a_r, br_r, c_r, b1_r, w2_r, b2_r,
             h_r, ss_r, sq_r):
        h = _dot(xt_r[...], a_r[...]) + _dot(ag_r[...], br_r[...])
        h += _dot(xu_r[...], c_r[...]) + b1_r[...]
        y = _dot(_lrelu(h), w2_r[...]) + b2_r[...]
        h_r[...] = y
        _acc_stats(y, ss_r, sq_r)

    return pl.pallas_call(
        body, grid=(_GN,),
        in_specs=[_rows(_BN, _D), _rows(_BN, 256), _full((1, _D)),
                  _full(w1a.shape), _full(w1b.shape), _full(w1c.shape),
                  _full(b1.shape), _full(w2.shape), _full(b2.shape)],
        out_specs=[_rows(_BN, _D), _full((1, _D)), _full((1, _D))],
        out_shape=[jax.ShapeDtypeStruct((_N, _D), f32),
                   jax.ShapeDtypeStruct((1, _D), f32),
                   jax.ShapeDtypeStruct((1, _D), f32)],
    )(xt, agg, xu, w1a, w1b, w1c, b1, w2, b2)


def _tc_global(xu, cs, ct, w1a, w1b, w1c, b1, w2, b2, rg):
    def body(xu_r, cs_r, ct_r, a_r, br_r, c_r, b1_r, w2_r, b2_r, rg_r, o_r):
        ms = cs_r[...] * (1.0 / _N)
        mt = ct_r[...] * (1.0 / _N)
        h = _dot(xu_r[...], a_r[...]) + _dot(ms, br_r[...])
        h += _dot(mt, c_r[...]) + b1_r[...]
        y = _dot(_lrelu(h), w2_r[...]) + b2_r[...]
        den = lax.rsqrt(jnp.mean(y * y, axis=-1, keepdims=True)
                        + jnp.finfo(jnp.float32).eps)
        o_r[...] = y * den * rg_r[...]

    return pl.pallas_call(
        body, grid=(1,),
        in_specs=[_full((1, _D))] * 3 + [_full(w1a.shape), _full(w1b.shape),
                  _full(w1c.shape), _full(b1.shape), _full(w2.shape),
                  _full(b2.shape), _full((1, _D))],
        out_specs=[_full((1, _D))],
        out_shape=[jax.ShapeDtypeStruct((1, _D), f32)],
    )(xu, cs, ct, w1a, w1b, w1c, b1, w2, b2, rg)


def _sc_gather_pair(x_s, x_t, src, tgt):
    return _build_gather(2)(x_s, x_t, src, tgt)


def _sc_gather_one(tab, idx):
    return _build_gather(1)(tab, idx)[0]


def _sc_stats(msg, src_pad):
    return _build_stats()(msg, src_pad)


def _sc_agg(msg_t, tgt_pad):
    return _build_agg()(msg_t, tgt_pad)[0]


def _pad_idx(v):
    r = v.reshape(_NS, _NCHS, _CE)
    r = jnp.pad(r, ((0, 0), (0, _IPAD - _NCHS), (0, 0)))
    return r.reshape(_NS * _IPAD, _CE)


def kernel(x_s, x_t, edge_index, edge_attr, x_u, params):
    p = params
    src = edge_index[0].astype(jnp.int32)
    tgt = edge_index[1].astype(jnp.int32)
    src_pad = _pad_idx(src)
    tgt_pad = _pad_idx(tgt)
    r2 = lambda a: a.reshape(1, -1)

    gs, gt = _sc_gather_pair(x_s, x_t, src, tgt)

    we1 = p['We1']
    h2, es, eq = _tc_edge_mlp(gs, gt, edge_attr, x_u,
                              we1[0:128], we1[128:256], we1[256:384],
                              we1[384:512], r2(p['be1']), p['We2'],
                              r2(p['be2']))

    wsm1 = p['Wsm1']
    ean, msg = _tc_bn_edge_msg(h2, gt, es, eq, r2(p['bne_g']), r2(p['bne_b']),
                               wsm1[:128], wsm1[128:], r2(p['bsm1']),
                               p['Wsm2'], r2(p['bsm2']))

    s1, s2, s3, s4, cnt = _sc_stats(msg, src_pad)
    mean, std, skew, kurt = _tc_moment_fin(s1, s2, s3, s4, cnt)

    wsu1 = p['Wsu1']
    ws = (wsu1[0:128], wsu1[128:384], wsu1[384:640], wsu1[640:896],
          wsu1[896:1152], wsu1[1152:1280])
    h_s, fs, fq = _tc_src_update(x_s, mean, std, skew, kurt, x_u,
                                 ws, r2(p['bsu1']), p['Wsu2'], r2(p['bsu2']))
    x_s_new, cs_sum = _tc_bn_rows(h_s, fs, fq, r2(p['bns_g']), r2(p['bns_b']),
                                  _N)

    gsn = _sc_gather_one(x_s_new, src)
    wtm1 = p['Wtm1']
    msg_t = _tc_tgt_msg(gsn, ean, wtm1[:128], wtm1[128:], r2(p['btm1']),
                        p['Wtm2'], r2(p['btm2']))[0]
    agg = _sc_agg(msg_t, tgt_pad)

    wtu1 = p['Wtu1']
    h_t, ts, tq = _tc_tgt_update(x_t, agg, x_u, wtu1[0:128], wtu1[128:384],
                                 wtu1[384:512], r2(p['btu1']), p['Wtu2'],
                                 r2(p['btu2']))
    x_t_new, ct_sum = _tc_bn_rows(h_t, ts, tq, r2(p['bnt_g']), r2(p['bnt_b']),
                                  _N)

    wg1 = p['Wg1']
    x_u_new = _tc_global(x_u, cs_sum, ct_sum, wg1[0:128], wg1[128:256],
                         wg1[256:384], r2(p['bg1']), p['Wg2'], r2(p['bg2']),
                         r2(p['rms_g']))[0]

    return (x_s_new, x_t_new, edge_index, ean, x_u_new)


# speedup vs baseline: 3.4909x; 1.0182x over previous
import functools

import jax
import jax.numpy as jnp
from jax import lax
from jax.experimental import pallas as pl
from jax.experimental.pallas import tpu as pltpu
from jax.experimental.pallas import tpu_sc as plsc

_N = 10000
_E = 160000
_D = 128
_SLOPE = 0.01
_NC, _NS, _L = 2, 16, 16
_NW = _NC * _NS

_CE = 80
_EPT = _E // _NS
_NCHS = _EPT // _CE
_IPAD = 128
_CZ = 80
_NZ = _N // _CZ

f32 = jnp.float32


def _lrelu(x):
    return jnp.where(x >= 0, x, _SLOPE * x)


def _mesh():
    return plsc.VectorSubcoreMesh(core_axis_name="c", subcore_axis_name="s")


def _fill(buf, nrows, ncols, val):
    def row(r, _):
        for g in range(ncols // _L):
            buf[r, pl.ds(g * _L, _L)] = jnp.full((_L,), val, f32)
        return 0
    lax.fori_loop(0, nrows, row, 0)


def _rr_chunks(sid, fn):
    for k in range(-(-_NZ // _NS)):
        j = sid + _NS * k
        pl.when(j < _NZ)(functools.partial(fn, j))


def _build_gather(n_tab):
    ew = _E // _NW
    c = 200
    nch = ew // c

    def body(*refs):
        tabs = refs[:n_tab]
        idxs = refs[n_tab:2 * n_tab]
        outs = refs[2 * n_tab:3 * n_tab]
        idx_v, r0, r1, sem0, sem1 = refs[3 * n_tab:]
        rows = (r0, r1)
        sems = (sem0, sem1)
        wid = lax.axis_index("s") * _NC + lax.axis_index("c")
        base = wid * ew
        for tab, ih, oh in zip(tabs, idxs, outs):
            pltpu.sync_copy(ih.at[pl.ds(base, ew)], idx_v)

            def g_at(k, b):
                pltpu.async_copy(tab.at[idx_v.at[pl.ds(k * c, c)]],
                                 rows[b], sems[b])

            def g_wait(b):
                pltpu.make_async_copy(tab.at[idx_v.at[pl.ds(0, c)]],
                                      rows[b], sems[b]).wait()

            g_at(0, 0)
            g_at(1, 1)

            def pair(k2, _):
                for b in range(2):
                    k = 2 * k2 + b
                    g_wait(b)
                    pltpu.sync_copy(rows[b], oh.at[pl.ds(base + k * c, c)])

                    @pl.when(k + 2 < nch)
                    def _():
                        g_at(k + 2, b)
                return 0
            lax.fori_loop(0, nch // 2, pair, 0)
            k = nch - 1
            g_wait(k % 2)
            pltpu.sync_copy(rows[k % 2], oh.at[pl.ds(base + k * c, c)])

    return pl.kernel(
        body,
        out_type=[jax.ShapeDtypeStruct((_E, _D), f32)] * n_tab,
        mesh=_mesh(),
        scratch_types=[
            pltpu.VMEM((ew,), jnp.int32),
            pltpu.VMEM((c, _D), f32),
            pltpu.VMEM((c, _D), f32),
            pltpu.SemaphoreType.DMA,
            pltpu.SemaphoreType.DMA,
        ],
    )


def _pow_into(kind, src, dst):
    def powrow(r, _):
        for g in range(128 // _L):
            v = src[r, pl.ds(g * _L, _L)]
            v2 = v * v
            if kind == "sq":
                dst[r, pl.ds(g * _L, _L)] = v2
            elif kind == "cube":
                dst[r, pl.ds(g * _L, _L)] = v2 * v
            else:
                dst[r, pl.ds(g * _L, _L)] = v2 * v2
        return 0
    lax.fori_loop(0, _CE, powrow, 0)


def _stats_body(msg_h, srcp_h, s1_h, s2_h, s3_h, s4_h, cnt_h,
                idx_v, m0, m1, sbuf, sem0, sem1, acc):
    cid = lax.axis_index("c")
    sid = lax.axis_index("s")
    mb = (m0, m1)
    sems = (sem0, sem1)
    pltpu.sync_copy(srcp_h.at[pl.ds(sid * _IPAD, _IPAD)], idx_v)

    def one_pass(f0, kind):
        _fill(m0, _CZ, 128, 0.0)

        def zero(j):
            pltpu.sync_copy(m0, acc.at[pl.ds(j * _CZ, _CZ)])
        _rr_chunks(sid, zero)
        if kind == "cnt":
            _fill(sbuf, _CE, 128, 1.0)
        plsc.subcore_barrier()

        def ld_at(j, b):
            e0 = sid * _EPT + j * _CE
            pltpu.async_copy(msg_h.at[pl.ds(e0, _CE), pl.ds(f0, 128)],
                             mb[b], sems[b])

        def ld_wait(b):
            pltpu.make_async_copy(
                msg_h.at[pl.ds(0, _CE), pl.ds(f0, 128)], mb[b],
                sems[b]).wait()

        if kind == "cnt":
            def step(j, _):
                pltpu.sync_copy(sbuf, acc.at[idx_v.at[j]], add=True)
                return 0
            lax.fori_loop(0, _NCHS, step, 0)
        else:
            ld_at(0, 0)
            ld_at(1, 1)

            def pair(k, _):
                for b in range(2):
                    j = 2 * k + b
                    ld_wait(b)
                    if kind == "sum":
                        pltpu.sync_copy(mb[b], acc.at[idx_v.at[j]], add=True)
                    else:
                        _pow_into(kind, mb[b], sbuf)
                        pltpu.sync_copy(sbuf, acc.at[idx_v.at[j]], add=True)

                    @pl.when(j + 2 < _NCHS)
                    def _():
                        ld_at(j + 2, b)
                return 0
            lax.fori_loop(0, _NCHS // 2, pair, 0)
            j = _NCHS - 1
            ld_wait(j % 2)
            if kind == "sum":
                pltpu.sync_copy(mb[j % 2], acc.at[idx_v.at[j]], add=True)
            else:
                _pow_into(kind, mb[j % 2], sbuf)
                pltpu.sync_copy(sbuf, acc.at[idx_v.at[j]], add=True)
        plsc.subcore_barrier()

        out_h = {"sum": s1_h, "sq": s2_h, "cube": s3_h, "quart": s4_h,
                 "cnt": cnt_h}[kind]

        def read(j):
            r0 = j * _CZ
            pltpu.sync_copy(acc.at[pl.ds(r0, _CZ)], m0)
            if kind == "cnt":
                pltpu.sync_copy(m0, out_h.at[pl.ds(r0, _CZ)])
            else:
                pltpu.sync_copy(m0,
                                out_h.at[pl.ds(r0, _CZ), pl.ds(f0, 128)])
        _rr_chunks(sid, read)
        plsc.subcore_barrier()

    for f0, core, kind in ((0, 0, "sum"), (128, 0, "sum"),
                           (0, 0, "cube"), (128, 0, "cube"), (0, 0, "cnt"),
                           (0, 1, "sq"), (128, 1, "sq"),
                           (0, 1, "quart"), (128, 1, "quart")):
        pl.when(cid == core)(functools.partial(one_pass, f0, kind))


def _build_stats():
    return pl.kernel(
        _stats_body,
        out_type=[jax.ShapeDtypeStruct((_N, 256), f32),
                  jax.ShapeDtypeStruct((_N, 256), f32),
                  jax.ShapeDtypeStruct((_N, 256), f32),
                  jax.ShapeDtypeStruct((_N, 256), f32),
                  jax.ShapeDtypeStruct((_N, 128), f32)],
        mesh=_mesh(),
        scratch_types=[
            pltpu.VMEM((_IPAD, _CE), jnp.int32),
            pltpu.VMEM((_CE, 128), f32),
            pltpu.VMEM((_CE, 128), f32),
            pltpu.VMEM((_CE, 128), f32),
            pltpu.SemaphoreType.DMA,
            pltpu.SemaphoreType.DMA,
            pltpu.VMEM_SHARED((_N, 128), f32),
        ],
    )


def _agg_body(msg_h, tgtp_h, agg_h, idx_v, m0, m1, sem0, sem1, acc):
    cid = lax.axis_index("c")
    sid = lax.axis_index("s")
    mb = (m0, m1)
    sems = (sem0, sem1)
    pltpu.sync_copy(tgtp_h.at[pl.ds(sid * _IPAD, _IPAD)], idx_v)

    def one_pass(f0):
        _fill(m0, _CZ, 128, 0.0)

        def zero(j):
            pltpu.sync_copy(m0, acc.at[pl.ds(j * _CZ, _CZ)])
        _rr_chunks(sid, zero)
        plsc.subcore_barrier()

        def ld_at(j, b):
            e0 = sid * _EPT + j * _CE
            pltpu.async_copy(msg_h.at[pl.ds(e0, _CE), pl.ds(f0, 128)],
                             mb[b], sems[b])

        def ld_wait(b):
            pltpu.make_async_copy(
                msg_h.at[pl.ds(0, _CE), pl.ds(f0, 128)], mb[b],
                sems[b]).wait()

        ld_at(0, 0)
        ld_at(1, 1)

        def pair(k, _):
            for b in range(2):
                j = 2 * k + b
                ld_wait(b)
                pltpu.sync_copy(mb[b], acc.at[idx_v.at[j]], add=True)

                @pl.when(j + 2 < _NCHS)
                def _():
                    ld_at(j + 2, b)
            return 0
        lax.fori_loop(0, _NCHS // 2, pair, 0)
        j = _NCHS - 1
        ld_wait(j % 2)
        pltpu.sync_copy(mb[j % 2], acc.at[idx_v.at[j]], add=True)
        plsc.subcore_barrier()

        def read(j):
            r0 = j * _CZ
            pltpu.sync_copy(acc.at[pl.ds(r0, _CZ)], m0)
            pltpu.sync_copy(m0, agg_h.at[pl.ds(r0, _CZ), pl.ds(f0, 128)])
        _rr_chunks(sid, read)
        plsc.subcore_barrier()

    pl.when(cid == 0)(functools.partial(one_pass, 0))
    pl.when(cid == 1)(functools.partial(one_pass, 128))


def _build_agg():
    return pl.kernel(
        _agg_body,
        out_type=[jax.ShapeDtypeStruct((_N, 256), f32)],
        mesh=_mesh(),
        scratch_types=[
            pltpu.VMEM((_IPAD, _CE), jnp.int32),
            pltpu.VMEM((_CE, 128), f32),
            pltpu.VMEM((_CE, 128), f32),
            pltpu.SemaphoreType.DMA,
            pltpu.SemaphoreType.DMA,
            pltpu.VMEM_SHARED((_N, 128), f32),
        ],
    )


_BE = 2000
_GE = _E // _BE
_BN = 1000
_GN = _N // _BN


def _full(shape):
    return pl.BlockSpec(shape, lambda i: (0,) * len(shape))


def _rows(b, w):
    return pl.BlockSpec((b, w), lambda i: (i, 0))


def _dot(a, b):
    return jnp.dot(a, b, preferred_element_type=f32)


def _acc_stats(y, ss_r, sq_r):
    @pl.when(pl.program_id(0) == 0)
    def _():
        ss_r[...] = jnp.zeros_like(ss_r)
        sq_r[...] = jnp.zeros_like(sq_r)
    ss_r[...] += jnp.sum(y, 0, keepdims=True)
    sq_r[...] += jnp.sum(y * y, 0, keepdims=True)


def _tc_edge_mlp(gs, gt, ea, xu, w1a, w1b, w1c, w1d, b1, w2, b2):
    def body(gs_r, gt_r, ea_r, xu_r, a_r, br_r, c_r, d_r, b1_r, w2_r, b2_r,
             h2_r, ss_r, sq_r):
        h = _dot(gs_r[...], a_r[...]) + _dot(gt_r[...], br_r[...])
        h += _dot(ea_r[...], c_r[...])
        h += _dot(xu_r[...], d_r[...]) + b1_r[...]
        y = _dot(_lrelu(h), w2_r[...]) + b2_r[...]
        h2_r[...] = y
        _acc_stats(y, ss_r, sq_r)

    return pl.pallas_call(
        body, grid=(_GE,),
        in_specs=[_rows(_BE, _D)] * 3 + [_full(xu.shape), _full(w1a.shape),
                  _full(w1b.shape), _full(w1c.shape), _full(w1d.shape),
                  _full(b1.shape), _full(w2.shape), _full(b2.shape)],
        out_specs=[_rows(_BE, _D), _full((1, _D)), _full((1, _D))],
        out_shape=[jax.ShapeDtypeStruct((_E, _D), f32),
                   jax.ShapeDtypeStruct((1, _D), f32),
                   jax.ShapeDtypeStruct((1, _D), f32)],
    )(gs, gt, ea, xu, w1a, w1b, w1c, w1d, b1, w2, b2)


def _tc_bn_edge_msg(h2, gt, ss, sq, g, b, w1a, w1b, b1, w2, b2):
    def body(h2_r, gt_r, ss_r, sq_r, g_r, b_r, a_r, br_r, b1_r, w2_r, b2_r,
             ean_r, msg_r):
        m = ss_r[...] * (1.0 / _E)
        v = sq_r[...] * (1.0 / _E) - m * m
        inv = lax.rsqrt(v + 1e-5)
        ean = (h2_r[...] - m) * inv * g_r[...] + b_r[...]
        ean_r[...] = ean
        h = _dot(gt_r[...], a_r[...]) + _dot(ean, br_r[...]) + b1_r[...]
        msg_r[...] = _dot(_lrelu(h), w2_r[...]) + b2_r[...]

    return pl.pallas_call(
        body, grid=(_GE,),
        in_specs=[_rows(_BE, _D)] * 2 + [_full((1, _D))] * 4 +
                 [_full(w1a.shape), _full(w1b.shape), _full(b1.shape),
                  _full(w2.shape), _full(b2.shape)],
        out_specs=[_rows(_BE, _D), _rows(_BE, 256)],
        out_shape=[jax.ShapeDtypeStruct((_E, _D), f32),
                   jax.ShapeDtypeStruct((_E, 256), f32)],
    )(h2, gt, ss, sq, g, b, w1a, w1b, b1, w2, b2)


def _moments_from_sums(s1, s2, s3, s4, c):
    cm = jnp.maximum(c, 1.0)
    m1 = s1 / cm
    m2 = s2 / cm
    m3 = s3 / cm
    m4 = s4 / cm
    var = _lrelu(m2 - m1 * m1)
    std = jnp.sqrt(var + 1e-6)
    m1sq = m1 * m1
    c3 = m3 - 3.0 * m1 * m2 + 2.0 * m1sq * m1
    c4 = m4 - 4.0 * m1 * m3 + 6.0 * m1sq * m2 - 3.0 * m1sq * m1sq
    std2 = std * std
    return m1, std, c3 / (std2 * std), c4 / (std2 * std2)


def _tc_src_update(xs, s1, s2, s3, s4, cnt, xu, ws, b1, w2, b2):
    def body(xs_r, s1_r, s2_r, s3_r, s4_r, c_r, xu_r,
             w0, w1, w2_, w3, w4, w5, b1_r, wo, b2_r, h_r, ss_r, sq_r):
        mean, std, skew, kurt = _moments_from_sums(
            s1_r[...], s2_r[...], s3_r[...], s4_r[...], c_r[:, 0:1])
        h = _dot(xs_r[...], w0[...]) + _dot(mean, w1[...])
        h += _dot(std, w2_[...]) + _dot(skew, w3[...])
        h += _dot(kurt, w4[...])
        h += _dot(xu_r[...], w5[...]) + b1_r[...]
        y = _dot(_lrelu(h), wo[...]) + b2_r[...]
        h_r[...] = y
        _acc_stats(y, ss_r, sq_r)

    return pl.pallas_call(
        body, grid=(_GN,),
        in_specs=[_rows(_BN, _D), _rows(_BN, 256), _rows(_BN, 256),
                  _rows(_BN, 256), _rows(_BN, 256), _rows(_BN, 128),
                  _full((1, _D))] +
                 [_full(w.shape) for w in ws] +
                 [_full(b1.shape), _full(w2.shape), _full(b2.shape)],
        out_specs=[_rows(_BN, _D), _full((1, _D)), _full((1, _D))],
        out_shape=[jax.ShapeDtypeStruct((_N, _D), f32),
                   jax.ShapeDtypeStruct((1, _D), f32),
                   jax.ShapeDtypeStruct((1, _D), f32)],
    )(xs, s1, s2, s3, s4, cnt, xu, *ws, b1, w2, b2)


def _tc_bn_rows(h, ss, sq, g, b, nrows):
    def body(h_r, ss_r, sq_r, g_r, b_r, xn_r, cs_r):
        m = ss_r[...] * (1.0 / nrows)
        v = sq_r[...] * (1.0 / nrows) - m * m
        xn = (h_r[...] - m) * lax.rsqrt(v + 1e-5) * g_r[...] + b_r[...]
        xn_r[...] = xn

        @pl.when(pl.program_id(0) == 0)
        def _():
            cs_r[...] = jnp.zeros_like(cs_r)
        cs_r[...] += jnp.sum(xn, 0, keepdims=True)

    return pl.pallas_call(
        body, grid=(_GN,),
        in_specs=[_rows(_BN, _D)] + [_full((1, _D))] * 4,
        out_specs=[_rows(_BN, _D), _full((1, _D))],
        out_shape=[jax.ShapeDtypeStruct((_N, _D), f32),
                   jax.ShapeDtypeStruct((1, _D), f32)],
    )(h, ss, sq, g, b)


def _tc_tgt_msg(gsn, ean, w1a, w1b, b1, w2, b2):
    def body(gs_r, ea_r, a_r, br_r, b1_r, w2_r, b2_r, msg_r):
        h = _dot(gs_r[...], a_r[...]) + _dot(ea_r[...], br_r[...]) + b1_r[...]
        msg_r[...] = _dot(_lrelu(h), w2_r[...]) + b2_r[...]

    return pl.pallas_call(
        body, grid=(_GE,),
        in_specs=[_rows(_BE, _D)] * 2 + [_full(w1a.shape), _full(w1b.shape),
                  _full(b1.shape), _full(w2.shape), _full(b2.shape)],
        out_specs=[_rows(_BE, 256)],
        out_shape=[jax.ShapeDtypeStruct((_E, 256), f32)],
    )(gsn, ean, w1a, w1b, b1, w2, b2)


def _tc_tgt_update(xt, agg, xu, w1a, w1b, w1c, b1, w2, b2):
    def body(xt_r, ag_r, xu_r, a_r, br_r, c_r, b1_r, w2_r, b2_r,
             h_r, ss_r, sq_r):
        h = _dot(xt_r[...], a_r[...]) + _dot(ag_r[...], br_r[...])
        h += _dot(xu_r[...], c_r[...]) + b1_r[...]
        y = _dot(_lrelu(h), w2_r[...]) + b2_r[...]
        h_r[...] = y
        _acc_stats(y, ss_r, sq_r)

    return pl.pallas_call(
        body, grid=(_GN,),
        in_specs=[_rows(_BN, _D), _rows(_BN, 256), _full((1, _D)),
                  _full(w1a.shape), _full(w1b.shape), _full(w1c.shape),
                  _full(b1.shape), _full(w2.shape), _full(b2.shape)],
        out_specs=[_rows(_BN, _D), _full((1, _D)), _full((1, _D))],
        out_shape=[jax.ShapeDtypeStruct((_N, _D), f32),
                   jax.ShapeDtypeStruct((1, _D), f32),
                   jax.ShapeDtypeStruct((1, _D), f32)],
    )(xt, agg, xu, w1a, w1b, w1c, b1, w2, b2)


def _tc_global(xu, cs, ct, w1a, w1b, w1c, b1, w2, b2, rg):
    def body(xu_r, cs_r, ct_r, a_r, br_r, c_r, b1_r, w2_r, b2_r, rg_r, o_r):
        ms = cs_r[...] * (1.0 / _N)
        mt = ct_r[...] * (1.0 / _N)
        h = _dot(xu_r[...], a_r[...]) + _dot(ms, br_r[...])
        h += _dot(mt, c_r[...]) + b1_r[...]
        y = _dot(_lrelu(h), w2_r[...]) + b2_r[...]
        den = lax.rsqrt(jnp.mean(y * y, axis=-1, keepdims=True)
                        + jnp.finfo(jnp.float32).eps)
        o_r[...] = y * den * rg_r[...]

    return pl.pallas_call(
        body, grid=(1,),
        in_specs=[_full((1, _D))] * 3 + [_full(w1a.shape), _full(w1b.shape),
                  _full(w1c.shape), _full(b1.shape), _full(w2.shape),
                  _full(b2.shape), _full((1, _D))],
        out_specs=[_full((1, _D))],
        out_shape=[jax.ShapeDtypeStruct((1, _D), f32)],
    )(xu, cs, ct, w1a, w1b, w1c, b1, w2, b2, rg)


def _sc_gather_pair(x_s, x_t, src, tgt):
    return _build_gather(2)(x_s, x_t, src, tgt)


def _sc_gather_one(tab, idx):
    return _build_gather(1)(tab, idx)[0]


def _sc_stats(msg, src_pad):
    return _build_stats()(msg, src_pad)


def _sc_agg(msg_t, tgt_pad):
    return _build_agg()(msg_t, tgt_pad)[0]


def _pad_idx(v):
    r = v.reshape(_NS, _NCHS, _CE)
    r = jnp.pad(r, ((0, 0), (0, _IPAD - _NCHS), (0, 0)))
    return r.reshape(_NS * _IPAD, _CE)


def kernel(x_s, x_t, edge_index, edge_attr, x_u, params):
    p = params
    src = edge_index[0].astype(jnp.int32)
    tgt = edge_index[1].astype(jnp.int32)
    src_pad = _pad_idx(src)
    tgt_pad = _pad_idx(tgt)
    r2 = lambda a: a.reshape(1, -1)

    gs, gt = _sc_gather_pair(x_s, x_t, src, tgt)

    we1 = p['We1']
    h2, es, eq = _tc_edge_mlp(gs, gt, edge_attr, x_u,
                              we1[0:128], we1[128:256], we1[256:384],
                              we1[384:512], r2(p['be1']), p['We2'],
                              r2(p['be2']))

    wsm1 = p['Wsm1']
    ean, msg = _tc_bn_edge_msg(h2, gt, es, eq, r2(p['bne_g']), r2(p['bne_b']),
                               wsm1[:128], wsm1[128:], r2(p['bsm1']),
                               p['Wsm2'], r2(p['bsm2']))

    s1, s2, s3, s4, cnt = _sc_stats(msg, src_pad)

    wsu1 = p['Wsu1']
    ws = (wsu1[0:128], wsu1[128:384], wsu1[384:640], wsu1[640:896],
          wsu1[896:1152], wsu1[1152:1280])
    h_s, fs, fq = _tc_src_update(x_s, s1, s2, s3, s4, cnt, x_u,
                                 ws, r2(p['bsu1']), p['Wsu2'], r2(p['bsu2']))
    x_s_new, cs_sum = _tc_bn_rows(h_s, fs, fq, r2(p['bns_g']), r2(p['bns_b']),
                                  _N)

    gsn = _sc_gather_one(x_s_new, src)
    wtm1 = p['Wtm1']
    msg_t = _tc_tgt_msg(gsn, ean, wtm1[:128], wtm1[128:], r2(p['btm1']),
                        p['Wtm2'], r2(p['btm2']))[0]
    agg = _sc_agg(msg_t, tgt_pad)

    wtu1 = p['Wtu1']
    h_t, ts, tq = _tc_tgt_update(x_t, agg, x_u, wtu1[0:128], wtu1[128:384],
                                 wtu1[384:512], r2(p['btu1']), p['Wtu2'],
                                 r2(p['btu2']))
    x_t_new, ct_sum = _tc_bn_rows(h_t, ts, tq, r2(p['bnt_g']), r2(p['bnt_b']),
                                  _N)

    wg1 = p['Wg1']
    x_u_new = _tc_global(x_u, cs_sum, ct_sum, wg1[0:128], wg1[128:256],
                         wg1[256:384], r2(p['bg1']), p['Wg2'], r2(p['bg2']),
                         r2(p['rms_g']))[0]

    return (x_s_new, x_t_new, edge_index, ean, x_u_new)


# speedup vs baseline: 3.5856x; 1.0271x over previous
import functools

import jax
import jax.numpy as jnp
from jax import lax
from jax.experimental import pallas as pl
from jax.experimental.pallas import tpu as pltpu
from jax.experimental.pallas import tpu_sc as plsc

_N = 10000
_E = 160000
_D = 128
_SLOPE = 0.01
_NC, _NS, _L = 2, 16, 16
_NW = _NC * _NS

_CE = 80
_EPT = _E // _NS
_NCHS = _EPT // _CE
_IPAD = 128
_CZ = 80
_NZ = _N // _CZ

f32 = jnp.float32


def _lrelu(x):
    return jnp.where(x >= 0, x, _SLOPE * x)


def _mesh():
    return plsc.VectorSubcoreMesh(core_axis_name="c", subcore_axis_name="s")


def _fill(buf, nrows, ncols, val):
    def row(r, _):
        for g in range(ncols // _L):
            buf[r, pl.ds(g * _L, _L)] = jnp.full((_L,), val, f32)
        return 0
    lax.fori_loop(0, nrows, row, 0)


def _rr_chunks(sid, fn):
    for k in range(-(-_NZ // _NS)):
        j = sid + _NS * k
        pl.when(j < _NZ)(functools.partial(fn, j))


def _build_gather(n_tab):
    ew = _E // _NW
    c = 200
    nch = ew // c

    def body(*refs):
        tabs = refs[:n_tab]
        idxs = refs[n_tab:2 * n_tab]
        outs = refs[2 * n_tab:3 * n_tab]
        idx_v, r0, r1, sem0, sem1 = refs[3 * n_tab:]
        rows = (r0, r1)
        sems = (sem0, sem1)
        wid = lax.axis_index("s") * _NC + lax.axis_index("c")
        base = wid * ew
        for tab, ih, oh in zip(tabs, idxs, outs):
            pltpu.sync_copy(ih.at[pl.ds(base, ew)], idx_v)

            def g_at(k, b):
                pltpu.async_copy(tab.at[idx_v.at[pl.ds(k * c, c)]],
                                 rows[b], sems[b])

            def g_wait(b):
                pltpu.make_async_copy(tab.at[idx_v.at[pl.ds(0, c)]],
                                      rows[b], sems[b]).wait()

            g_at(0, 0)
            g_at(1, 1)

            def pair(k2, _):
                for b in range(2):
                    k = 2 * k2 + b
                    g_wait(b)
                    pltpu.sync_copy(rows[b], oh.at[pl.ds(base + k * c, c)])

                    @pl.when(k + 2 < nch)
                    def _():
                        g_at(k + 2, b)
                return 0
            lax.fori_loop(0, nch // 2, pair, 0)
            k = nch - 1
            g_wait(k % 2)
            pltpu.sync_copy(rows[k % 2], oh.at[pl.ds(base + k * c, c)])

    return pl.kernel(
        body,
        out_type=[jax.ShapeDtypeStruct((_E, _D), f32)] * n_tab,
        mesh=_mesh(),
        scratch_types=[
            pltpu.VMEM((ew,), jnp.int32),
            pltpu.VMEM((c, _D), f32),
            pltpu.VMEM((c, _D), f32),
            pltpu.SemaphoreType.DMA,
            pltpu.SemaphoreType.DMA,
        ],
    )


def _pow_into(kind, src, dst):
    def powrow(r, _):
        for g in range(128 // _L):
            v = src[r, pl.ds(g * _L, _L)]
            v2 = v * v
            if kind == "sq":
                dst[r, pl.ds(g * _L, _L)] = v2
            elif kind == "cube":
                dst[r, pl.ds(g * _L, _L)] = v2 * v
            else:
                dst[r, pl.ds(g * _L, _L)] = v2 * v2
        return 0
    lax.fori_loop(0, _CE, powrow, 0)


def _stats_body(msg_h, srcp_h, s1_h, s2_h, s3_h, s4_h, cnt_h,
                idx_v, m0, m1, sbuf, sem0, sem1, sem2, sem3, sem4, sem5,
                acc):
    cid = lax.axis_index("c")
    sid = lax.axis_index("s")
    mb = (m0, m1)
    sems = (sem0, sem1)
    pltpu.sync_copy(srcp_h.at[pl.ds(sid * _IPAD, _IPAD)], idx_v)

    def one_pass(f0, kind):
        _fill(m0, _CZ, 128, 0.0)

        def zero(j):
            pltpu.sync_copy(m0, acc.at[pl.ds(j * _CZ, _CZ)])
        _rr_chunks(sid, zero)
        if kind == "cnt":
            _fill(sbuf, _CE, 128, 1.0)
        plsc.subcore_barrier()

        def ld_at(j, b):
            e0 = sid * _EPT + j * _CE
            pltpu.async_copy(msg_h.at[pl.ds(e0, _CE), pl.ds(f0, 128)],
                             mb[b], sems[b])

        def ld_wait(b):
            pltpu.make_async_copy(
                msg_h.at[pl.ds(0, _CE), pl.ds(f0, 128)], mb[b],
                sems[b]).wait()

        if kind == "cnt":
            def step(j, _):
                pltpu.async_copy(sbuf, acc.at[idx_v.at[j]], sem0, add=True)
                return 0
            lax.fori_loop(0, _NCHS, step, 0)

            def drain(j, _):
                pltpu.make_async_copy(sbuf, acc.at[idx_v.at[0]],
                                      sem0).wait()
                return 0
            lax.fori_loop(0, _NCHS, drain, 0)
        elif kind == "sum":
            b3 = (m0, m1, sbuf)
            ls = (sem0, sem1, sem2)
            ss = (sem3, sem4, sem5)

            def ld3(j, b):
                e0 = sid * _EPT + j * _CE
                pltpu.async_copy(msg_h.at[pl.ds(e0, _CE), pl.ds(f0, 128)],
                                 b3[b], ls[b])

            def ld3_wait(b):
                pltpu.make_async_copy(
                    msg_h.at[pl.ds(0, _CE), pl.ds(f0, 128)], b3[b],
                    ls[b]).wait()

            def sc_wait(b):
                pltpu.make_async_copy(b3[b], acc.at[idx_v.at[0]],
                                      ss[b]).wait()

            for b in range(3):
                ld3(b, b)

            def triple(k, _):
                j0 = 3 * k
                for b in range(3):
                    ld3_wait(b)
                    pltpu.sync_copy(b3[b], acc.at[idx_v.at[j0 + b]],
                                    add=True)

                    @pl.when(j0 + b + 3 < _NCHS)
                    def _(b=b):
                        ld3(j0 + b + 3, b)
                return 0
            lax.fori_loop(0, _NCHS // 3, triple, 0)
            jt = (_NCHS // 3) * 3
            for j in range(jt, _NCHS):
                b = j - jt
                ld3_wait(b)
                pltpu.sync_copy(b3[b], acc.at[idx_v.at[j]], add=True)
        else:
            ld_at(0, 0)
            ld_at(1, 1)

            def pair(k, _):
                for b in range(2):
                    j = 2 * k + b
                    ld_wait(b)
                    _pow_into(kind, mb[b], sbuf)
                    pltpu.sync_copy(sbuf, acc.at[idx_v.at[j]], add=True)

                    @pl.when(j + 2 < _NCHS)
                    def _():
                        ld_at(j + 2, b)
                return 0
            lax.fori_loop(0, _NCHS // 2, pair, 0)
            j = _NCHS - 1
            ld_wait(j % 2)
            _pow_into(kind, mb[j % 2], sbuf)
            pltpu.sync_copy(sbuf, acc.at[idx_v.at[j]], add=True)
        plsc.subcore_barrier()

        out_h = {"sum": s1_h, "sq": s2_h, "cube": s3_h, "quart": s4_h,
                 "cnt": cnt_h}[kind]

        def read(j):
            r0 = j * _CZ
            pltpu.sync_copy(acc.at[pl.ds(r0, _CZ)], m0)
            if kind == "cnt":
                pltpu.sync_copy(m0, out_h.at[pl.ds(r0, _CZ)])
            else:
                pltpu.sync_copy(m0,
                                out_h.at[pl.ds(r0, _CZ), pl.ds(f0, 128)])
        _rr_chunks(sid, read)
        plsc.subcore_barrier()

    for f0, core, kind in ((0, 0, "sum"), (128, 0, "sum"),
                           (0, 0, "cube"), (128, 0, "cube"), (0, 0, "cnt"),
                           (0, 1, "sq"), (128, 1, "sq"),
                           (0, 1, "quart"), (128, 1, "quart")):
        pl.when(cid == core)(functools.partial(one_pass, f0, kind))


def _build_stats():
    return pl.kernel(
        _stats_body,
        out_type=[jax.ShapeDtypeStruct((_N, 256), f32),
                  jax.ShapeDtypeStruct((_N, 256), f32),
                  jax.ShapeDtypeStruct((_N, 256), f32),
                  jax.ShapeDtypeStruct((_N, 256), f32),
                  jax.ShapeDtypeStruct((_N, 128), f32)],
        mesh=_mesh(),
        scratch_types=[
            pltpu.VMEM((_IPAD, _CE), jnp.int32),
            pltpu.VMEM((_CE, 128), f32),
            pltpu.VMEM((_CE, 128), f32),
            pltpu.VMEM((_CE, 128), f32),
            pltpu.SemaphoreType.DMA,
            pltpu.SemaphoreType.DMA,
            pltpu.SemaphoreType.DMA,
            pltpu.SemaphoreType.DMA,
            pltpu.SemaphoreType.DMA,
            pltpu.SemaphoreType.DMA,
            pltpu.VMEM_SHARED((_N, 128), f32),
        ],
    )


def _agg_body(msg_h, tgtp_h, agg_h, idx_v, m0, m1, m2,
              sem0, sem1, sem2, sem3, sem4, sem5, acc):
    cid = lax.axis_index("c")
    sid = lax.axis_index("s")
    b3 = (m0, m1, m2)
    ls = (sem0, sem1, sem2)
    ss = (sem3, sem4, sem5)
    pltpu.sync_copy(tgtp_h.at[pl.ds(sid * _IPAD, _IPAD)], idx_v)

    def one_pass(f0):
        _fill(m0, _CZ, 128, 0.0)

        def zero(j):
            pltpu.sync_copy(m0, acc.at[pl.ds(j * _CZ, _CZ)])
        _rr_chunks(sid, zero)
        plsc.subcore_barrier()

        def ld3(j, b):
            e0 = sid * _EPT + j * _CE
            pltpu.async_copy(msg_h.at[pl.ds(e0, _CE), pl.ds(f0, 128)],
                             b3[b], ls[b])

        def ld3_wait(b):
            pltpu.make_async_copy(
                msg_h.at[pl.ds(0, _CE), pl.ds(f0, 128)], b3[b],
                ls[b]).wait()

        def sc_wait(b):
            pltpu.make_async_copy(b3[b], acc.at[idx_v.at[0]], ss[b]).wait()

        for b in range(3):
            ld3(b, b)

        def triple(k, _):
            j0 = 3 * k
            for b in range(3):
                ld3_wait(b)
                pltpu.sync_copy(b3[b], acc.at[idx_v.at[j0 + b]], add=True)

                @pl.when(j0 + b + 3 < _NCHS)
                def _(b=b):
                    ld3(j0 + b + 3, b)
            return 0
        lax.fori_loop(0, _NCHS // 3, triple, 0)
        jt = (_NCHS // 3) * 3
        for j in range(jt, _NCHS):
            b = j - jt
            ld3_wait(b)
            pltpu.sync_copy(b3[b], acc.at[idx_v.at[j]], add=True)
        plsc.subcore_barrier()

        def read(j):
            r0 = j * _CZ
            pltpu.sync_copy(acc.at[pl.ds(r0, _CZ)], m0)
            pltpu.sync_copy(m0, agg_h.at[pl.ds(r0, _CZ), pl.ds(f0, 128)])
        _rr_chunks(sid, read)
        plsc.subcore_barrier()

    pl.when(cid == 0)(functools.partial(one_pass, 0))
    pl.when(cid == 1)(functools.partial(one_pass, 128))


def _build_agg():
    return pl.kernel(
        _agg_body,
        out_type=[jax.ShapeDtypeStruct((_N, 256), f32)],
        mesh=_mesh(),
        scratch_types=[
            pltpu.VMEM((_IPAD, _CE), jnp.int32),
            pltpu.VMEM((_CE, 128), f32),
            pltpu.VMEM((_CE, 128), f32),
            pltpu.VMEM((_CE, 128), f32),
            pltpu.SemaphoreType.DMA,
            pltpu.SemaphoreType.DMA,
            pltpu.SemaphoreType.DMA,
            pltpu.SemaphoreType.DMA,
            pltpu.SemaphoreType.DMA,
            pltpu.SemaphoreType.DMA,
            pltpu.VMEM_SHARED((_N, 128), f32),
        ],
    )


_BE = 2000
_GE = _E // _BE
_BN = 1000
_GN = _N // _BN


def _full(shape):
    return pl.BlockSpec(shape, lambda i: (0,) * len(shape))


def _rows(b, w):
    return pl.BlockSpec((b, w), lambda i: (i, 0))


def _dot(a, b):
    return jnp.dot(a, b, preferred_element_type=f32)


def _acc_stats(y, ss_r, sq_r):
    @pl.when(pl.program_id(0) == 0)
    def _():
        ss_r[...] = jnp.zeros_like(ss_r)
        sq_r[...] = jnp.zeros_like(sq_r)
    ss_r[...] += jnp.sum(y, 0, keepdims=True)
    sq_r[...] += jnp.sum(y * y, 0, keepdims=True)


def _tc_edge_mlp(gs, gt, ea, xu, w1a, w1b, w1c, w1d, b1, w2, b2):
    def body(gs_r, gt_r, ea_r, xu_r, a_r, br_r, c_r, d_r, b1_r, w2_r, b2_r,
             h2_r, ss_r, sq_r):
        h = _dot(gs_r[...], a_r[...]) + _dot(gt_r[...], br_r[...])
        h += _dot(ea_r[...], c_r[...])
        h += _dot(xu_r[...], d_r[...]) + b1_r[...]
        y = _dot(_lrelu(h), w2_r[...]) + b2_r[...]
        h2_r[...] = y
        _acc_stats(y, ss_r, sq_r)

    return pl.pallas_call(
        body, grid=(_GE,),
        in_specs=[_rows(_BE, _D)] * 3 + [_full(xu.shape), _full(w1a.shape),
                  _full(w1b.shape), _full(w1c.shape), _full(w1d.shape),
                  _full(b1.shape), _full(w2.shape), _full(b2.shape)],
        out_specs=[_rows(_BE, _D), _full((1, _D)), _full((1, _D))],
        out_shape=[jax.ShapeDtypeStruct((_E, _D), f32),
                   jax.ShapeDtypeStruct((1, _D), f32),
                   jax.ShapeDtypeStruct((1, _D), f32)],
    )(gs, gt, ea, xu, w1a, w1b, w1c, w1d, b1, w2, b2)


def _tc_bn_edge_msg(h2, gt, ss, sq, g, b, w1a, w1b, b1, w2, b2):
    def body(h2_r, gt_r, ss_r, sq_r, g_r, b_r, a_r, br_r, b1_r, w2_r, b2_r,
             ean_r, msg_r):
        m = ss_r[...] * (1.0 / _E)
        v = sq_r[...] * (1.0 / _E) - m * m
        inv = lax.rsqrt(v + 1e-5)
        ean = (h2_r[...] - m) * inv * g_r[...] + b_r[...]
        ean_r[...] = ean
        h = _dot(gt_r[...], a_r[...]) + _dot(ean, br_r[...]) + b1_r[...]
        msg_r[...] = _dot(_lrelu(h), w2_r[...]) + b2_r[...]

    return pl.pallas_call(
        body, grid=(_GE,),
        in_specs=[_rows(_BE, _D)] * 2 + [_full((1, _D))] * 4 +
                 [_full(w1a.shape), _full(w1b.shape), _full(b1.shape),
                  _full(w2.shape), _full(b2.shape)],
        out_specs=[_rows(_BE, _D), _rows(_BE, 256)],
        out_shape=[jax.ShapeDtypeStruct((_E, _D), f32),
                   jax.ShapeDtypeStruct((_E, 256), f32)],
    )(h2, gt, ss, sq, g, b, w1a, w1b, b1, w2, b2)


def _moments_from_sums(s1, s2, s3, s4, c):
    cm = jnp.maximum(c, 1.0)
    m1 = s1 / cm
    m2 = s2 / cm
    m3 = s3 / cm
    m4 = s4 / cm
    var = _lrelu(m2 - m1 * m1)
    std = jnp.sqrt(var + 1e-6)
    m1sq = m1 * m1
    c3 = m3 - 3.0 * m1 * m2 + 2.0 * m1sq * m1
    c4 = m4 - 4.0 * m1 * m3 + 6.0 * m1sq * m2 - 3.0 * m1sq * m1sq
    std2 = std * std
    return m1, std, c3 / (std2 * std), c4 / (std2 * std2)


def _tc_src_update(xs, s1, s2, s3, s4, cnt, xu, ws, b1, w2, b2):
    def body(xs_r, s1_r, s2_r, s3_r, s4_r, c_r, xu_r,
             w0, w1, w2_, w3, w4, w5, b1_r, wo, b2_r, h_r, ss_r, sq_r):
        mean, std, skew, kurt = _moments_from_sums(
            s1_r[...], s2_r[...], s3_r[...], s4_r[...], c_r[:, 0:1])
        h = _dot(xs_r[...], w0[...]) + _dot(mean, w1[...])
        h += _dot(std, w2_[...]) + _dot(skew, w3[...])
        h += _dot(kurt, w4[...])
        h += _dot(xu_r[...], w5[...]) + b1_r[...]
        y = _dot(_lrelu(h), wo[...]) + b2_r[...]
        h_r[...] = y
        _acc_stats(y, ss_r, sq_r)

    return pl.pallas_call(
        body, grid=(_GN,),
        in_specs=[_rows(_BN, _D), _rows(_BN, 256), _rows(_BN, 256),
                  _rows(_BN, 256), _rows(_BN, 256), _rows(_BN, 128),
                  _full((1, _D))] +
                 [_full(w.shape) for w in ws] +
                 [_full(b1.shape), _full(w2.shape), _full(b2.shape)],
        out_specs=[_rows(_BN, _D), _full((1, _D)), _full((1, _D))],
        out_shape=[jax.ShapeDtypeStruct((_N, _D), f32),
                   jax.ShapeDtypeStruct((1, _D), f32),
                   jax.ShapeDtypeStruct((1, _D), f32)],
    )(xs, s1, s2, s3, s4, cnt, xu, *ws, b1, w2, b2)


def _tc_bn_rows(h, ss, sq, g, b, nrows):
    def body(h_r, ss_r, sq_r, g_r, b_r, xn_r, cs_r):
        m = ss_r[...] * (1.0 / nrows)
        v = sq_r[...] * (1.0 / nrows) - m * m
        xn = (h_r[...] - m) * lax.rsqrt(v + 1e-5) * g_r[...] + b_r[...]
        xn_r[...] = xn

        @pl.when(pl.program_id(0) == 0)
        def _():
            cs_r[...] = jnp.zeros_like(cs_r)
        cs_r[...] += jnp.sum(xn, 0, keepdims=True)

    return pl.pallas_call(
        body, grid=(_GN,),
        in_specs=[_rows(_BN, _D)] + [_full((1, _D))] * 4,
        out_specs=[_rows(_BN, _D), _full((1, _D))],
        out_shape=[jax.ShapeDtypeStruct((_N, _D), f32),
                   jax.ShapeDtypeStruct((1, _D), f32)],
    )(h, ss, sq, g, b)


def _tc_tgt_msg(gsn, ean, w1a, w1b, b1, w2, b2):
    def body(gs_r, ea_r, a_r, br_r, b1_r, w2_r, b2_r, msg_r):
        h = _dot(gs_r[...], a_r[...]) + _dot(ea_r[...], br_r[...]) + b1_r[...]
        msg_r[...] = _dot(_lrelu(h), w2_r[...]) + b2_r[...]

    return pl.pallas_call(
        body, grid=(_GE,),
        in_specs=[_rows(_BE, _D)] * 2 + [_full(w1a.shape), _full(w1b.shape),
                  _full(b1.shape), _full(w2.shape), _full(b2.shape)],
        out_specs=[_rows(_BE, 256)],
        out_shape=[jax.ShapeDtypeStruct((_E, 256), f32)],
    )(gsn, ean, w1a, w1b, b1, w2, b2)


def _tc_tgt_update(xt, agg, xu, w1a, w1b, w1c, b1, w2, b2):
    def body(xt_r, ag_r, xu_r, a_r, br_r, c_r, b1_r, w2_r, b2_r,
             h_r, ss_r, sq_r):
        h = _dot(xt_r[...], a_r[...]) + _dot(ag_r[...], br_r[...])
        h += _dot(xu_r[...], c_r[...]) + b1_r[...]
        y = _dot(_lrelu(h), w2_r[...]) + b2_r[...]
        h_r[...] = y
        _acc_stats(y, ss_r, sq_r)

    return pl.pallas_call(
        body, grid=(_GN,),
        in_specs=[_rows(_BN, _D), _rows(_BN, 256), _full((1, _D)),
                  _full(w1a.shape), _full(w1b.shape), _full(w1c.shape),
                  _full(b1.shape), _full(w2.shape), _full(b2.shape)],
        out_specs=[_rows(_BN, _D), _full((1, _D)), _full((1, _D))],
        out_shape=[jax.ShapeDtypeStruct((_N, _D), f32),
                   jax.ShapeDtypeStruct((1, _D), f32),
                   jax.ShapeDtypeStruct((1, _D), f32)],
    )(xt, agg, xu, w1a, w1b, w1c, b1, w2, b2)


def _tc_global(xu, cs, ct, w1a, w1b, w1c, b1, w2, b2, rg):
    def body(xu_r, cs_r, ct_r, a_r, br_r, c_r, b1_r, w2_r, b2_r, rg_r, o_r):
        ms = cs_r[...] * (1.0 / _N)
        mt = ct_r[...] * (1.0 / _N)
        h = _dot(xu_r[...], a_r[...]) + _dot(ms, br_r[...])
        h += _dot(mt, c_r[...]) + b1_r[...]
        y = _dot(_lrelu(h), w2_r[...]) + b2_r[...]
        den = lax.rsqrt(jnp.mean(y * y, axis=-1, keepdims=True)
                        + jnp.finfo(jnp.float32).eps)
        o_r[...] = y * den * rg_r[...]

    return pl.pallas_call(
        body, grid=(1,),
        in_specs=[_full((1, _D))] * 3 + [_full(w1a.shape), _full(w1b.shape),
                  _full(w1c.shape), _full(b1.shape), _full(w2.shape),
                  _full(b2.shape), _full((1, _D))],
        out_specs=[_full((1, _D))],
        out_shape=[jax.ShapeDtypeStruct((1, _D), f32)],
    )(xu, cs, ct, w1a, w1b, w1c, b1, w2, b2, rg)


def _sc_gather_pair(x_s, x_t, src, tgt):
    return _build_gather(2)(x_s, x_t, src, tgt)


def _sc_gather_one(tab, idx):
    return _build_gather(1)(tab, idx)[0]


def _sc_stats(msg, src_pad):
    return _build_stats()(msg, src_pad)


def _sc_agg(msg_t, tgt_pad):
    return _build_agg()(msg_t, tgt_pad)[0]


def _pad_idx(v):
    r = v.reshape(_NS, _NCHS, _CE)
    r = jnp.pad(r, ((0, 0), (0, _IPAD - _NCHS), (0, 0)))
    return r.reshape(_NS * _IPAD, _CE)


def kernel(x_s, x_t, edge_index, edge_attr, x_u, params):
    p = params
    src = edge_index[0].astype(jnp.int32)
    tgt = edge_index[1].astype(jnp.int32)
    src_pad = _pad_idx(src)
    tgt_pad = _pad_idx(tgt)
    r2 = lambda a: a.reshape(1, -1)

    gs, gt = _sc_gather_pair(x_s, x_t, src, tgt)

    we1 = p['We1']
    h2, es, eq = _tc_edge_mlp(gs, gt, edge_attr, x_u,
                              we1[0:128], we1[128:256], we1[256:384],
                              we1[384:512], r2(p['be1']), p['We2'],
                              r2(p['be2']))

    wsm1 = p['Wsm1']
    ean, msg = _tc_bn_edge_msg(h2, gt, es, eq, r2(p['bne_g']), r2(p['bne_b']),
                               wsm1[:128], wsm1[128:], r2(p['bsm1']),
                               p['Wsm2'], r2(p['bsm2']))

    s1, s2, s3, s4, cnt = _sc_stats(msg, src_pad)

    wsu1 = p['Wsu1']
    ws = (wsu1[0:128], wsu1[128:384], wsu1[384:640], wsu1[640:896],
          wsu1[896:1152], wsu1[1152:1280])
    h_s, fs, fq = _tc_src_update(x_s, s1, s2, s3, s4, cnt, x_u,
                                 ws, r2(p['bsu1']), p['Wsu2'], r2(p['bsu2']))
    x_s_new, cs_sum = _tc_bn_rows(h_s, fs, fq, r2(p['bns_g']), r2(p['bns_b']),
                                  _N)

    gsn = _sc_gather_one(x_s_new, src)
    wtm1 = p['Wtm1']
    msg_t = _tc_tgt_msg(gsn, ean, wtm1[:128], wtm1[128:], r2(p['btm1']),
                        p['Wtm2'], r2(p['btm2']))[0]
    agg = _sc_agg(msg_t, tgt_pad)

    wtu1 = p['Wtu1']
    h_t, ts, tq = _tc_tgt_update(x_t, agg, x_u, wtu1[0:128], wtu1[128:384],
                                 wtu1[384:512], r2(p['btu1']), p['Wtu2'],
                                 r2(p['btu2']))
    x_t_new, ct_sum = _tc_bn_rows(h_t, ts, tq, r2(p['bnt_g']), r2(p['bnt_b']),
                                  _N)

    wg1 = p['Wg1']
    x_u_new = _tc_global(x_u, cs_sum, ct_sum, wg1[0:128], wg1[128:256],
                         wg1[256:384], r2(p['bg1']), p['Wg2'], r2(p['bg2']),
                         r2(p['rms_g']))[0]

    return (x_s_new, x_t_new, edge_index, ean, x_u_new)


# speedup vs baseline: 3.7939x; 1.0581x over previous
import functools

import jax
import jax.numpy as jnp
from jax import lax
from jax.experimental import pallas as pl
from jax.experimental.pallas import tpu as pltpu
from jax.experimental.pallas import tpu_sc as plsc

_N = 10000
_E = 160000
_D = 128
_SLOPE = 0.01
_NC, _NS, _L = 2, 16, 16
_NW = _NC * _NS

_CE = 80
_EPT = _E // _NS
_NCHS = _EPT // _CE
_IPAD = 128
_CZ = 80
_NZ = _N // _CZ

f32 = jnp.float32


def _lrelu(x):
    return jnp.where(x >= 0, x, _SLOPE * x)


def _mesh():
    return plsc.VectorSubcoreMesh(core_axis_name="c", subcore_axis_name="s")


def _fill(buf, nrows, ncols, val):
    def row(r, _):
        for g in range(ncols // _L):
            buf[r, pl.ds(g * _L, _L)] = jnp.full((_L,), val, f32)
        return 0
    lax.fori_loop(0, nrows, row, 0)


def _rr_chunks(sid, fn):
    for k in range(-(-_NZ // _NS)):
        j = sid + _NS * k
        pl.when(j < _NZ)(functools.partial(fn, j))


def _build_gather(n_tab):
    ew = _E // _NW
    c = 200
    nch = ew // c

    def body(*refs):
        tabs = refs[:n_tab]
        idxs = refs[n_tab:2 * n_tab]
        outs = refs[2 * n_tab:3 * n_tab]
        idx_v, r0, r1, sem0, sem1 = refs[3 * n_tab:]
        rows = (r0, r1)
        sems = (sem0, sem1)
        wid = lax.axis_index("s") * _NC + lax.axis_index("c")
        base = wid * ew
        for tab, ih, oh in zip(tabs, idxs, outs):
            pltpu.sync_copy(ih.at[pl.ds(base, ew)], idx_v)

            def g_at(k, b):
                pltpu.async_copy(tab.at[idx_v.at[pl.ds(k * c, c)]],
                                 rows[b], sems[b])

            def g_wait(b):
                pltpu.make_async_copy(tab.at[idx_v.at[pl.ds(0, c)]],
                                      rows[b], sems[b]).wait()

            g_at(0, 0)
            g_at(1, 1)

            def pair(k2, _):
                for b in range(2):
                    k = 2 * k2 + b
                    g_wait(b)
                    pltpu.sync_copy(rows[b], oh.at[pl.ds(base + k * c, c)])

                    @pl.when(k + 2 < nch)
                    def _():
                        g_at(k + 2, b)
                return 0
            lax.fori_loop(0, nch // 2, pair, 0)
            k = nch - 1
            g_wait(k % 2)
            pltpu.sync_copy(rows[k % 2], oh.at[pl.ds(base + k * c, c)])

    return pl.kernel(
        body,
        out_type=[jax.ShapeDtypeStruct((_E, _D), f32)] * n_tab,
        mesh=_mesh(),
        scratch_types=[
            pltpu.VMEM((ew,), jnp.int32),
            pltpu.VMEM((c, _D), f32),
            pltpu.VMEM((c, _D), f32),
            pltpu.SemaphoreType.DMA,
            pltpu.SemaphoreType.DMA,
        ],
    )


def _pow_into(kind, src, dst):
    def powrow(r, _):
        for g in range(128 // _L):
            v = src[r, pl.ds(g * _L, _L)]
            v2 = v * v
            if kind == "sq":
                dst[r, pl.ds(g * _L, _L)] = v2
            elif kind == "cube":
                dst[r, pl.ds(g * _L, _L)] = v2 * v
            else:
                dst[r, pl.ds(g * _L, _L)] = v2 * v2
        return 0
    lax.fori_loop(0, _CE, powrow, 0)


def _stats_body(msg_h, srcp_h, s1_h, s2_h, s3_h, s4_h, cnt_h,
                idx_v, m0, m1, sbuf, sem0, sem1, sem2, sem3, sem4, sem5,
                acc):
    cid = lax.axis_index("c")
    sid = lax.axis_index("s")
    mb = (m0, m1)
    sems = (sem0, sem1)
    pltpu.sync_copy(srcp_h.at[pl.ds(sid * _IPAD, _IPAD)], idx_v)

    def one_pass(f0, kind):
        _fill(m0, _CZ, 128, 0.0)

        def zero(j):
            pltpu.sync_copy(m0, acc.at[pl.ds(j * _CZ, _CZ)])
        _rr_chunks(sid, zero)
        if kind == "cnt":
            _fill(sbuf, _CE, 128, 1.0)
        plsc.subcore_barrier()

        def ld_at(j, b):
            e0 = sid * _EPT + j * _CE
            pltpu.async_copy(msg_h.at[pl.ds(e0, _CE), pl.ds(f0, 128)],
                             mb[b], sems[b])

        def ld_wait(b):
            pltpu.make_async_copy(
                msg_h.at[pl.ds(0, _CE), pl.ds(f0, 128)], mb[b],
                sems[b]).wait()

        if kind == "cnt":
            def step(j, _):
                pltpu.async_copy(sbuf, acc.at[idx_v.at[j]], sem0, add=True)
                return 0
            lax.fori_loop(0, _NCHS, step, 0)

            def drain(j, _):
                pltpu.make_async_copy(sbuf, acc.at[idx_v.at[0]],
                                      sem0).wait()
                return 0
            lax.fori_loop(0, _NCHS, drain, 0)
        elif kind == "sum":
            b3 = (m0, m1, sbuf)
            ls = (sem0, sem1, sem2)
            ss = (sem3, sem4, sem5)

            def ld3(j, b):
                e0 = sid * _EPT + j * _CE
                pltpu.async_copy(msg_h.at[pl.ds(e0, _CE), pl.ds(f0, 128)],
                                 b3[b], ls[b])

            def ld3_wait(b):
                pltpu.make_async_copy(
                    msg_h.at[pl.ds(0, _CE), pl.ds(f0, 128)], b3[b],
                    ls[b]).wait()

            def sc_wait(b):
                pltpu.make_async_copy(b3[b], acc.at[idx_v.at[0]],
                                      ss[b]).wait()

            for b in range(3):
                ld3(b, b)

            def triple(k, _):
                j0 = 3 * k
                for b in range(3):
                    ld3_wait(b)
                    pltpu.sync_copy(b3[b], acc.at[idx_v.at[j0 + b]],
                                    add=True)

                    @pl.when(j0 + b + 3 < _NCHS)
                    def _(b=b):
                        ld3(j0 + b + 3, b)
                return 0
            lax.fori_loop(0, _NCHS // 3, triple, 0)
            jt = (_NCHS // 3) * 3
            for j in range(jt, _NCHS):
                b = j - jt
                ld3_wait(b)
                pltpu.sync_copy(b3[b], acc.at[idx_v.at[j]], add=True)
        else:
            ld_at(0, 0)
            ld_at(1, 1)

            def pair(k, _):
                for b in range(2):
                    j = 2 * k + b
                    ld_wait(b)
                    _pow_into(kind, mb[b], sbuf)
                    pltpu.sync_copy(sbuf, acc.at[idx_v.at[j]], add=True)

                    @pl.when(j + 2 < _NCHS)
                    def _():
                        ld_at(j + 2, b)
                return 0
            lax.fori_loop(0, _NCHS // 2, pair, 0)
            j = _NCHS - 1
            ld_wait(j % 2)
            _pow_into(kind, mb[j % 2], sbuf)
            pltpu.sync_copy(sbuf, acc.at[idx_v.at[j]], add=True)
        plsc.subcore_barrier()

        out_h = {"sum": s1_h, "sq": s2_h, "cube": s3_h, "quart": s4_h,
                 "cnt": cnt_h}[kind]

        def read(j):
            r0 = j * _CZ
            pltpu.sync_copy(acc.at[pl.ds(r0, _CZ)], m0)
            if kind == "cnt":
                pltpu.sync_copy(m0, out_h.at[pl.ds(r0, _CZ)])
            else:
                pltpu.sync_copy(m0,
                                out_h.at[pl.ds(r0, _CZ), pl.ds(f0, 128)])
        _rr_chunks(sid, read)
        plsc.subcore_barrier()

    for f0, core, kind in ((0, 0, "sum"), (128, 0, "sum"),
                           (0, 0, "cube"), (128, 0, "cube"), (0, 0, "cnt"),
                           (0, 1, "sq"), (128, 1, "sq"),
                           (0, 1, "quart"), (128, 1, "quart")):
        pl.when(cid == core)(functools.partial(one_pass, f0, kind))


def _build_stats():
    return pl.kernel(
        _stats_body,
        out_type=[jax.ShapeDtypeStruct((_N, 256), f32),
                  jax.ShapeDtypeStruct((_N, 256), f32),
                  jax.ShapeDtypeStruct((_N, 256), f32),
                  jax.ShapeDtypeStruct((_N, 256), f32),
                  jax.ShapeDtypeStruct((_N, 128), f32)],
        mesh=_mesh(),
        scratch_types=[
            pltpu.VMEM((_IPAD, _CE), jnp.int32),
            pltpu.VMEM((_CE, 128), f32),
            pltpu.VMEM((_CE, 128), f32),
            pltpu.VMEM((_CE, 128), f32),
            pltpu.SemaphoreType.DMA,
            pltpu.SemaphoreType.DMA,
            pltpu.SemaphoreType.DMA,
            pltpu.SemaphoreType.DMA,
            pltpu.SemaphoreType.DMA,
            pltpu.SemaphoreType.DMA,
            pltpu.VMEM_SHARED((_N, 128), f32),
        ],
    )


def _agg_body(msg_h, tgtp_h, agg_h, idx_v, m0, m1, m2,
              sem0, sem1, sem2, sem3, sem4, sem5, acc):
    cid = lax.axis_index("c")
    sid = lax.axis_index("s")
    b3 = (m0, m1, m2)
    ls = (sem0, sem1, sem2)
    ss = (sem3, sem4, sem5)
    pltpu.sync_copy(tgtp_h.at[pl.ds(sid * _IPAD, _IPAD)], idx_v)

    def one_pass(f0):
        _fill(m0, _CZ, 128, 0.0)

        def zero(j):
            pltpu.sync_copy(m0, acc.at[pl.ds(j * _CZ, _CZ)])
        _rr_chunks(sid, zero)
        plsc.subcore_barrier()

        def ld3(j, b):
            e0 = sid * _EPT + j * _CE
            pltpu.async_copy(msg_h.at[pl.ds(e0, _CE), pl.ds(f0, 128)],
                             b3[b], ls[b])

        def ld3_wait(b):
            pltpu.make_async_copy(
                msg_h.at[pl.ds(0, _CE), pl.ds(f0, 128)], b3[b],
                ls[b]).wait()

        def sc_wait(b):
            pltpu.make_async_copy(b3[b], acc.at[idx_v.at[0]], ss[b]).wait()

        for b in range(3):
            ld3(b, b)

        def triple(k, _):
            j0 = 3 * k
            for b in range(3):
                ld3_wait(b)
                pltpu.sync_copy(b3[b], acc.at[idx_v.at[j0 + b]], add=True)

                @pl.when(j0 + b + 3 < _NCHS)
                def _(b=b):
                    ld3(j0 + b + 3, b)
            return 0
        lax.fori_loop(0, _NCHS // 3, triple, 0)
        jt = (_NCHS // 3) * 3
        for j in range(jt, _NCHS):
            b = j - jt
            ld3_wait(b)
            pltpu.sync_copy(b3[b], acc.at[idx_v.at[j]], add=True)
        plsc.subcore_barrier()

        def read(j):
            r0 = j * _CZ
            pltpu.sync_copy(acc.at[pl.ds(r0, _CZ)], m0)
            pltpu.sync_copy(m0, agg_h.at[pl.ds(r0, _CZ), pl.ds(f0, 128)])
        _rr_chunks(sid, read)
        plsc.subcore_barrier()

    pl.when(cid == 0)(functools.partial(one_pass, 0))
    pl.when(cid == 1)(functools.partial(one_pass, 128))


def _build_agg():
    return pl.kernel(
        _agg_body,
        out_type=[jax.ShapeDtypeStruct((_N, 256), f32)],
        mesh=_mesh(),
        scratch_types=[
            pltpu.VMEM((_IPAD, _CE), jnp.int32),
            pltpu.VMEM((_CE, 128), f32),
            pltpu.VMEM((_CE, 128), f32),
            pltpu.VMEM((_CE, 128), f32),
            pltpu.SemaphoreType.DMA,
            pltpu.SemaphoreType.DMA,
            pltpu.SemaphoreType.DMA,
            pltpu.SemaphoreType.DMA,
            pltpu.SemaphoreType.DMA,
            pltpu.SemaphoreType.DMA,
            pltpu.VMEM_SHARED((_N, 128), f32),
        ],
    )


_BE = 4000
_GE = _E // _BE
_BN = 2000
_GN = _N // _BN


def _full(shape):
    return pl.BlockSpec(shape, lambda i: (0,) * len(shape))


def _rows(b, w):
    return pl.BlockSpec((b, w), lambda i: (i, 0))


def _dot(a, b):
    return jnp.dot(a, b, preferred_element_type=f32)


def _acc_stats(y, ss_r, sq_r):
    @pl.when(pl.program_id(0) == 0)
    def _():
        ss_r[...] = jnp.zeros_like(ss_r)
        sq_r[...] = jnp.zeros_like(sq_r)
    ss_r[...] += jnp.sum(y, 0, keepdims=True)
    sq_r[...] += jnp.sum(y * y, 0, keepdims=True)


def _tc_edge_mlp(gs, gt, ea, xu, w1a, w1b, w1c, w1d, b1, w2, b2):
    def body(gs_r, gt_r, ea_r, xu_r, a_r, br_r, c_r, d_r, b1_r, w2_r, b2_r,
             h2_r, ss_r, sq_r):
        h = _dot(gs_r[...], a_r[...]) + _dot(gt_r[...], br_r[...])
        h += _dot(ea_r[...], c_r[...])
        h += _dot(xu_r[...], d_r[...]) + b1_r[...]
        y = _dot(_lrelu(h), w2_r[...]) + b2_r[...]
        h2_r[...] = y
        _acc_stats(y, ss_r, sq_r)

    return pl.pallas_call(
        body, grid=(_GE,),
        in_specs=[_rows(_BE, _D)] * 3 + [_full(xu.shape), _full(w1a.shape),
                  _full(w1b.shape), _full(w1c.shape), _full(w1d.shape),
                  _full(b1.shape), _full(w2.shape), _full(b2.shape)],
        out_specs=[_rows(_BE, _D), _full((1, _D)), _full((1, _D))],
        out_shape=[jax.ShapeDtypeStruct((_E, _D), f32),
                   jax.ShapeDtypeStruct((1, _D), f32),
                   jax.ShapeDtypeStruct((1, _D), f32)],
    )(gs, gt, ea, xu, w1a, w1b, w1c, w1d, b1, w2, b2)


def _tc_bn_edge_msg(h2, gt, ss, sq, g, b, w1a, w1b, b1, w2, b2):
    def body(h2_r, gt_r, ss_r, sq_r, g_r, b_r, a_r, br_r, b1_r, w2_r, b2_r,
             ean_r, msg_r):
        m = ss_r[...] * (1.0 / _E)
        v = sq_r[...] * (1.0 / _E) - m * m
        inv = lax.rsqrt(v + 1e-5)
        ean = (h2_r[...] - m) * inv * g_r[...] + b_r[...]
        ean_r[...] = ean
        h = _dot(gt_r[...], a_r[...]) + _dot(ean, br_r[...]) + b1_r[...]
        msg_r[...] = _dot(_lrelu(h), w2_r[...]) + b2_r[...]

    return pl.pallas_call(
        body, grid=(_GE,),
        in_specs=[_rows(_BE, _D)] * 2 + [_full((1, _D))] * 4 +
                 [_full(w1a.shape), _full(w1b.shape), _full(b1.shape),
                  _full(w2.shape), _full(b2.shape)],
        out_specs=[_rows(_BE, _D), _rows(_BE, 256)],
        out_shape=[jax.ShapeDtypeStruct((_E, _D), f32),
                   jax.ShapeDtypeStruct((_E, 256), f32)],
    )(h2, gt, ss, sq, g, b, w1a, w1b, b1, w2, b2)


def _moments_from_sums(s1, s2, s3, s4, c):
    cm = jnp.maximum(c, 1.0)
    m1 = s1 / cm
    m2 = s2 / cm
    m3 = s3 / cm
    m4 = s4 / cm
    var = _lrelu(m2 - m1 * m1)
    std = jnp.sqrt(var + 1e-6)
    m1sq = m1 * m1
    c3 = m3 - 3.0 * m1 * m2 + 2.0 * m1sq * m1
    c4 = m4 - 4.0 * m1 * m3 + 6.0 * m1sq * m2 - 3.0 * m1sq * m1sq
    std2 = std * std
    return m1, std, c3 / (std2 * std), c4 / (std2 * std2)


def _tc_src_update(xs, s1, s2, s3, s4, cnt, xu, ws, b1, w2, b2):
    def body(xs_r, s1_r, s2_r, s3_r, s4_r, c_r, xu_r,
             w0, w1, w2_, w3, w4, w5, b1_r, wo, b2_r, h_r, ss_r, sq_r):
        mean, std, skew, kurt = _moments_from_sums(
            s1_r[...], s2_r[...], s3_r[...], s4_r[...], c_r[:, 0:1])
        h = _dot(xs_r[...], w0[...]) + _dot(mean, w1[...])
        h += _dot(std, w2_[...]) + _dot(skew, w3[...])
        h += _dot(kurt, w4[...])
        h += _dot(xu_r[...], w5[...]) + b1_r[...]
        y = _dot(_lrelu(h), wo[...]) + b2_r[...]
        h_r[...] = y
        _acc_stats(y, ss_r, sq_r)

    return pl.pallas_call(
        body, grid=(_GN,),
        in_specs=[_rows(_BN, _D), _rows(_BN, 256), _rows(_BN, 256),
                  _rows(_BN, 256), _rows(_BN, 256), _rows(_BN, 128),
                  _full((1, _D))] +
                 [_full(w.shape) for w in ws] +
                 [_full(b1.shape), _full(w2.shape), _full(b2.shape)],
        out_specs=[_rows(_BN, _D), _full((1, _D)), _full((1, _D))],
        out_shape=[jax.ShapeDtypeStruct((_N, _D), f32),
                   jax.ShapeDtypeStruct((1, _D), f32),
                   jax.ShapeDtypeStruct((1, _D), f32)],
    )(xs, s1, s2, s3, s4, cnt, xu, *ws, b1, w2, b2)


def _tc_bn_rows(h, ss, sq, g, b, nrows):
    def body(h_r, ss_r, sq_r, g_r, b_r, xn_r, cs_r):
        m = ss_r[...] * (1.0 / nrows)
        v = sq_r[...] * (1.0 / nrows) - m * m
        xn = (h_r[...] - m) * lax.rsqrt(v + 1e-5) * g_r[...] + b_r[...]
        xn_r[...] = xn

        @pl.when(pl.program_id(0) == 0)
        def _():
            cs_r[...] = jnp.zeros_like(cs_r)
        cs_r[...] += jnp.sum(xn, 0, keepdims=True)

    return pl.pallas_call(
        body, grid=(_GN,),
        in_specs=[_rows(_BN, _D)] + [_full((1, _D))] * 4,
        out_specs=[_rows(_BN, _D), _full((1, _D))],
        out_shape=[jax.ShapeDtypeStruct((_N, _D), f32),
                   jax.ShapeDtypeStruct((1, _D), f32)],
    )(h, ss, sq, g, b)


def _tc_tgt_msg(gsn, ean, w1a, w1b, b1, w2, b2):
    def body(gs_r, ea_r, a_r, br_r, b1_r, w2_r, b2_r, msg_r):
        h = _dot(gs_r[...], a_r[...]) + _dot(ea_r[...], br_r[...]) + b1_r[...]
        msg_r[...] = _dot(_lrelu(h), w2_r[...]) + b2_r[...]

    return pl.pallas_call(
        body, grid=(_GE,),
        in_specs=[_rows(_BE, _D)] * 2 + [_full(w1a.shape), _full(w1b.shape),
                  _full(b1.shape), _full(w2.shape), _full(b2.shape)],
        out_specs=[_rows(_BE, 256)],
        out_shape=[jax.ShapeDtypeStruct((_E, 256), f32)],
    )(gsn, ean, w1a, w1b, b1, w2, b2)


def _tc_tgt_update(xt, agg, xu, w1a, w1b, w1c, b1, w2, b2):
    def body(xt_r, ag_r, xu_r, a_r, br_r, c_r, b1_r, w2_r, b2_r,
             h_r, ss_r, sq_r):
        h = _dot(xt_r[...], a_r[...]) + _dot(ag_r[...], br_r[...])
        h += _dot(xu_r[...], c_r[...]) + b1_r[...]
        y = _dot(_lrelu(h), w2_r[...]) + b2_r[...]
        h_r[...] = y
        _acc_stats(y, ss_r, sq_r)

    return pl.pallas_call(
        body, grid=(_GN,),
        in_specs=[_rows(_BN, _D), _rows(_BN, 256), _full((1, _D)),
                  _full(w1a.shape), _full(w1b.shape), _full(w1c.shape),
                  _full(b1.shape), _full(w2.shape), _full(b2.shape)],
        out_specs=[_rows(_BN, _D), _full((1, _D)), _full((1, _D))],
        out_shape=[jax.ShapeDtypeStruct((_N, _D), f32),
                   jax.ShapeDtypeStruct((1, _D), f32),
                   jax.ShapeDtypeStruct((1, _D), f32)],
    )(xt, agg, xu, w1a, w1b, w1c, b1, w2, b2)


def _tc_global(xu, cs, ct, w1a, w1b, w1c, b1, w2, b2, rg):
    def body(xu_r, cs_r, ct_r, a_r, br_r, c_r, b1_r, w2_r, b2_r, rg_r, o_r):
        ms = cs_r[...] * (1.0 / _N)
        mt = ct_r[...] * (1.0 / _N)
        h = _dot(xu_r[...], a_r[...]) + _dot(ms, br_r[...])
        h += _dot(mt, c_r[...]) + b1_r[...]
        y = _dot(_lrelu(h), w2_r[...]) + b2_r[...]
        den = lax.rsqrt(jnp.mean(y * y, axis=-1, keepdims=True)
                        + jnp.finfo(jnp.float32).eps)
        o_r[...] = y * den * rg_r[...]

    return pl.pallas_call(
        body, grid=(1,),
        in_specs=[_full((1, _D))] * 3 + [_full(w1a.shape), _full(w1b.shape),
                  _full(w1c.shape), _full(b1.shape), _full(w2.shape),
                  _full(b2.shape), _full((1, _D))],
        out_specs=[_full((1, _D))],
        out_shape=[jax.ShapeDtypeStruct((1, _D), f32)],
    )(xu, cs, ct, w1a, w1b, w1c, b1, w2, b2, rg)


def _sc_gather_pair(x_s, x_t, src, tgt):
    return _build_gather(2)(x_s, x_t, src, tgt)


def _sc_gather_one(tab, idx):
    return _build_gather(1)(tab, idx)[0]


def _sc_stats(msg, src_pad):
    return _build_stats()(msg, src_pad)


def _sc_agg(msg_t, tgt_pad):
    return _build_agg()(msg_t, tgt_pad)[0]


def _pad_idx(v):
    r = v.reshape(_NS, _NCHS, _CE)
    r = jnp.pad(r, ((0, 0), (0, _IPAD - _NCHS), (0, 0)))
    return r.reshape(_NS * _IPAD, _CE)


def kernel(x_s, x_t, edge_index, edge_attr, x_u, params):
    p = params
    src = edge_index[0].astype(jnp.int32)
    tgt = edge_index[1].astype(jnp.int32)
    src_pad = _pad_idx(src)
    tgt_pad = _pad_idx(tgt)
    r2 = lambda a: a.reshape(1, -1)

    gs, gt = _sc_gather_pair(x_s, x_t, src, tgt)

    we1 = p['We1']
    h2, es, eq = _tc_edge_mlp(gs, gt, edge_attr, x_u,
                              we1[0:128], we1[128:256], we1[256:384],
                              we1[384:512], r2(p['be1']), p['We2'],
                              r2(p['be2']))

    wsm1 = p['Wsm1']
    ean, msg = _tc_bn_edge_msg(h2, gt, es, eq, r2(p['bne_g']), r2(p['bne_b']),
                               wsm1[:128], wsm1[128:], r2(p['bsm1']),
                               p['Wsm2'], r2(p['bsm2']))

    s1, s2, s3, s4, cnt = _sc_stats(msg, src_pad)

    wsu1 = p['Wsu1']
    ws = (wsu1[0:128], wsu1[128:384], wsu1[384:640], wsu1[640:896],
          wsu1[896:1152], wsu1[1152:1280])
    h_s, fs, fq = _tc_src_update(x_s, s1, s2, s3, s4, cnt, x_u,
                                 ws, r2(p['bsu1']), p['Wsu2'], r2(p['bsu2']))
    x_s_new, cs_sum = _tc_bn_rows(h_s, fs, fq, r2(p['bns_g']), r2(p['bns_b']),
                                  _N)

    gsn = _sc_gather_one(x_s_new, src)
    wtm1 = p['Wtm1']
    msg_t = _tc_tgt_msg(gsn, ean, wtm1[:128], wtm1[128:], r2(p['btm1']),
                        p['Wtm2'], r2(p['btm2']))[0]
    agg = _sc_agg(msg_t, tgt_pad)

    wtu1 = p['Wtu1']
    h_t, ts, tq = _tc_tgt_update(x_t, agg, x_u, wtu1[0:128], wtu1[128:384],
                                 wtu1[384:512], r2(p['btu1']), p['Wtu2'],
                                 r2(p['btu2']))
    x_t_new, ct_sum = _tc_bn_rows(h_t, ts, tq, r2(p['bnt_g']), r2(p['bnt_b']),
                                  _N)

    wg1 = p['Wg1']
    x_u_new = _tc_global(x_u, cs_sum, ct_sum, wg1[0:128], wg1[128:256],
                         wg1[256:384], r2(p['bg1']), p['Wg2'], r2(p['bg2']),
                         r2(p['rms_g']))[0]

    return (x_s_new, x_t_new, edge_index, ean, x_u_new)
